# baseline (speedup 1.0000x reference)
import functools
import jax
import jax.numpy as jnp
import numpy as np
from jax import lax
from jax.experimental import pallas as pl
from jax.experimental.pallas import tpu as pltpu
from jax.experimental.pallas import tpu_sc as plsc

_NP = 10
_IMG = 256
_HALF = _IMG // 2 + 1
_NBINS = _IMG * _HALF
_FC = 256


def _dot(a, b):
    return jax.lax.dot(a, b, precision=jax.lax.Precision.HIGHEST,
                       preferred_element_type=jnp.float32)


def _dft_stats_kernel(x_ref, cm_ref, sm_ref, out_ref):
    x = x_ref[0]
    cm = cm_ref[...]
    sm = sm_ref[...]
    p = _dot(x, cm)
    q = _dot(x, sm)
    fre = _dot(cm, p) - _dot(sm, q)
    fim = -(_dot(cm, q) + _dot(sm, p))
    mag = jnp.sqrt(fre * fre + fim * fim)
    ang = jnp.arctan2(fim, fre)
    col = jax.lax.broadcasted_iota(jnp.int32, (_IMG, _IMG), 1)
    hmask = (col < _HALF).astype(jnp.float32)
    s1 = jnp.sum(mag * hmask)
    s2 = jnp.sum(ang * hmask)
    dc = jnp.sum(x)
    lane = jax.lax.broadcasted_iota(jnp.int32, (1, 128), 1)
    out_ref[0] = jnp.where(
        lane == 0, s1, jnp.where(lane == 1, s2, jnp.where(lane == 2, dc, 0.0)))


def _sigmoid_v(v):
    return 1.0 / (1.0 + jnp.exp(-v))


def _band_mask_v(featv, w1, b1, w2, b2):
    c1 = _sigmoid_v(featv * w1 + b1)
    c2 = _sigmoid_v(featv * w2 + b2)
    ind = jnp.where(c2 > c1, 1.0, 0.0)
    return jnp.where(c1 * ind < 1.0 / _IMG, 1.0, 0.0)


def _epilogue_sc_kernel(stats_hbm, wconst_hbm, out_hbm,
                        stats_v, wconst_v, pooled_v, row_v):
    info = plsc.get_sparse_core_info()
    nc = info.num_cores
    wid = lax.axis_index("s") * nc + lax.axis_index("c")

    pltpu.sync_copy(stats_hbm, stats_v)
    pltpu.sync_copy(wconst_hbm, wconst_v)
    _HD = 256

    inv = jnp.float32(1.0 / (_IMG * _IMG))
    zeros16 = jnp.zeros((16,), jnp.int32)

    dnums = lax.GatherDimensionNumbers(
        offset_dims=(), collapsed_slice_dims=(0,), start_index_map=(0,))

    def splat(v, i):
        return lax.gather(v, (zeros16 + i)[:, None], dnums, (1,),
                          mode=lax.GatherScatterMode.PROMISE_IN_BOUNDS)

    for bc in range(12):
        srow = stats_v[pl.ds(bc * 128, 16)]
        feat1 = splat(srow, 0) * (1.0 / _NBINS)
        feat2 = splat(srow, 1) * (1.0 / _NBINS)
        dcv = splat(srow, 2)
        masks = []
        for m, featv in ((0, feat1), (1, feat2)):
            mrow = lambda a, k: wconst_v[pl.ds(((m * 2 + a) * 4 + k) * 16, 16)]
            mx = _band_mask_v(featv, mrow(0, 0), mrow(0, 1),
                              mrow(0, 2), mrow(0, 3))
            my = _band_mask_v(featv, mrow(1, 0), mrow(1, 1),
                              mrow(1, 2), mrow(1, 3))
            masks.append(jnp.minimum(mx + my, 1.0))
        mask1, mask2 = masks
        amp = jnp.abs(dcv) * inv
        negf = jnp.where(dcv < 0.0, 1.0, 0.0)
        cos_d = 1.0 - 2.0 * negf * mask2
        cos_c = 1.0 - 2.0 * negf * (1.0 - mask2)
        pooled_v[pl.ds(bc * 16, 16)] = amp * mask1 * cos_d
        pooled_v[pl.ds((12 + bc) * 16, 16)] = amp * (1.0 - mask1) * cos_c

    def emit_row(r):
        b = r // _NP
        n = r % _NP
        for c in range(3):
            pdrow = pooled_v[pl.ds((b * 3 + c) * 16, 16)]
            pcrow = pooled_v[pl.ds((12 + b * 3 + c) * 16, 16)]
            pd = splat(pdrow, n)
            pc = splat(pcrow, n)
            for k in range(_FC // 16):
                ws = wconst_v[pl.ds(_HD + c * _FC + k * 16, 16)]
                wg = wconst_v[pl.ds(_HD + (4 + c) * _FC + k * 16, 16)]
                if c == 0:
                    bs = wconst_v[pl.ds(_HD + 3 * _FC + k * 16, 16)]
                    bg = wconst_v[pl.ds(_HD + 7 * _FC + k * 16, 16)]
                    row_v[pl.ds(k * 16, 16)] = bs + pc * ws
                    row_v[pl.ds(_FC + k * 16, 16)] = bg + pd * wg
                else:
                    row_v[pl.ds(k * 16, 16)] = (
                        row_v[pl.ds(k * 16, 16)] + pc * ws)
                    row_v[pl.ds(_FC + k * 16, 16)] = (
                        row_v[pl.ds(_FC + k * 16, 16)] + pd * wg)
        pltpu.sync_copy(row_v, out_hbm.at[r])

    emit_row(wid)

    @pl.when(wid < 8)
    def _():
        emit_row(wid + 32)


def kernel(x, W1, B1, W2, B2, Wsem, bsem, Wgen, bgen):
    B, C, W, H = x.shape
    xi = x.reshape(B * C, W, H)

    idx = jnp.arange(_IMG, dtype=jnp.int32)
    m = (idx[:, None] * idx[None, :]) % _IMG
    theta = (2.0 * np.pi / _IMG) * m.astype(jnp.float32)
    cm = jnp.cos(theta)
    sm = jnp.sin(theta)

    stats = pl.pallas_call(
        _dft_stats_kernel,
        grid=(B * C,),
        in_specs=[
            pl.BlockSpec((1, _IMG, _IMG), lambda i: (i, 0, 0)),
            pl.BlockSpec((_IMG, _IMG), lambda i: (0, 0)),
            pl.BlockSpec((_IMG, _IMG), lambda i: (0, 0)),
        ],
        out_specs=pl.BlockSpec((1, 1, 128), lambda i: (i, 0, 0)),
        out_shape=jax.ShapeDtypeStruct((B * C, 1, 128), jnp.float32),
    )(xi, cm, sm)
    stats_s = stats.reshape(-1)

    rows = []
    for Wm, Bm in ((W1, B1), (W2, B2)):
        for a in range(2):
            for arr in (Wm[a, 1], Bm[a, 1], Wm[a, 2], Bm[a, 2]):
                rows.append(jnp.pad(arr, (0, 16 - _NP)))
    wconst = jnp.concatenate(
        [jnp.stack(rows).reshape(-1), Wsem.reshape(-1), bsem,
         Wgen.reshape(-1), bgen])

    mesh = plsc.VectorSubcoreMesh(core_axis_name="c", subcore_axis_name="s")
    epilogue = pl.kernel(
        _epilogue_sc_kernel,
        mesh=mesh,
        out_type=jax.ShapeDtypeStruct((B * _NP, 2 * _FC), jnp.float32),
        scratch_types=[
            pltpu.VMEM((12 * 128,), jnp.float32),
            pltpu.VMEM((2304,), jnp.float32),
            pltpu.VMEM((24 * 16,), jnp.float32),
            pltpu.VMEM((2 * _FC,), jnp.float32),
        ],
    )
    return epilogue(stats_s, wconst)


# speedup vs baseline: 1.1659x; 1.1659x over previous
import functools
import jax
import jax.numpy as jnp
import numpy as np
from jax import lax
from jax.experimental import pallas as pl
from jax.experimental.pallas import tpu as pltpu
from jax.experimental.pallas import tpu_sc as plsc

_NP = 10
_IMG = 256
_HALF = _IMG // 2 + 1
_NBINS = _IMG * _HALF
_FC = 256


def _dot(a, b):
    return jax.lax.dot(a, b, precision=jax.lax.Precision.HIGHEST,
                       preferred_element_type=jnp.float32)


def _dft_stats_kernel(x_ref, cm_ref, sm_ref, out_ref):
    cm = cm_ref[...]
    sm = sm_ref[...]
    col = jax.lax.broadcasted_iota(jnp.int32, (_IMG, _IMG), 1)
    hmask = (col < _HALF).astype(jnp.float32)
    lane = jax.lax.broadcasted_iota(jnp.int32, (1, 128), 1)
    for j in range(2):
        x = x_ref[j]
        p = _dot(x, cm)
        q = _dot(x, sm)
        fre = _dot(cm, p) - _dot(sm, q)
        fim = -(_dot(cm, q) + _dot(sm, p))
        mag = jnp.sqrt(fre * fre + fim * fim)
        ang = jnp.arctan2(fim, fre)
        s1 = jnp.sum(mag * hmask)
        s2 = jnp.sum(ang * hmask)
        dc = jnp.sum(x)
        out_ref[j] = jnp.where(
            lane == 0, s1,
            jnp.where(lane == 1, s2, jnp.where(lane == 2, dc, 0.0)))


def _sigmoid_v(v):
    return 1.0 / (1.0 + jnp.exp(-v))


def _band_mask_v(featv, w1, b1, w2, b2):
    c1 = _sigmoid_v(featv * w1 + b1)
    c2 = _sigmoid_v(featv * w2 + b2)
    ind = jnp.where(c2 > c1, 1.0, 0.0)
    return jnp.where(c1 * ind < 1.0 / _IMG, 1.0, 0.0)


def _epilogue_sc_kernel(stats_hbm, wconst_hbm, out_hbm,
                        stats_v, wconst_v, pooled_v, row_v):
    info = plsc.get_sparse_core_info()
    nc = info.num_cores
    wid = lax.axis_index("s") * nc + lax.axis_index("c")

    pltpu.sync_copy(stats_hbm, stats_v)
    pltpu.sync_copy(wconst_hbm, wconst_v)
    _HD = 256

    inv = jnp.float32(1.0 / (_IMG * _IMG))
    zeros16 = jnp.zeros((16,), jnp.int32)

    dnums = lax.GatherDimensionNumbers(
        offset_dims=(), collapsed_slice_dims=(0,), start_index_map=(0,))

    def splat(v, i):
        return lax.gather(v, (zeros16 + i)[:, None], dnums, (1,),
                          mode=lax.GatherScatterMode.PROMISE_IN_BOUNDS)

    for bc in range(12):
        srow = stats_v[pl.ds(bc * 16, 16)]
        feat1 = splat(srow, 0) * (1.0 / _NBINS)
        feat2 = splat(srow, 1) * (1.0 / _NBINS)
        dcv = splat(srow, 2)
        masks = []
        for m, featv in ((0, feat1), (1, feat2)):
            mrow = lambda a, k: wconst_v[pl.ds(((m * 2 + a) * 4 + k) * 16, 16)]
            mx = _band_mask_v(featv, mrow(0, 0), mrow(0, 1),
                              mrow(0, 2), mrow(0, 3))
            my = _band_mask_v(featv, mrow(1, 0), mrow(1, 1),
                              mrow(1, 2), mrow(1, 3))
            masks.append(jnp.minimum(mx + my, 1.0))
        mask1, mask2 = masks
        amp = jnp.abs(dcv) * inv
        negf = jnp.where(dcv < 0.0, 1.0, 0.0)
        cos_d = 1.0 - 2.0 * negf * mask2
        cos_c = 1.0 - 2.0 * negf * (1.0 - mask2)
        pooled_v[pl.ds(bc * 16, 16)] = amp * mask1 * cos_d
        pooled_v[pl.ds((12 + bc) * 16, 16)] = amp * (1.0 - mask1) * cos_c

    def emit_row(r):
        b = r // _NP
        n = r % _NP
        for c in range(3):
            pdrow = pooled_v[pl.ds((b * 3 + c) * 16, 16)]
            pcrow = pooled_v[pl.ds((12 + b * 3 + c) * 16, 16)]
            pd = splat(pdrow, n)
            pc = splat(pcrow, n)
            for k in range(_FC // 16):
                ws = wconst_v[pl.ds(_HD + c * _FC + k * 16, 16)]
                wg = wconst_v[pl.ds(_HD + (4 + c) * _FC + k * 16, 16)]
                if c == 0:
                    bs = wconst_v[pl.ds(_HD + 3 * _FC + k * 16, 16)]
                    bg = wconst_v[pl.ds(_HD + 7 * _FC + k * 16, 16)]
                    row_v[pl.ds(k * 16, 16)] = bs + pc * ws
                    row_v[pl.ds(_FC + k * 16, 16)] = bg + pd * wg
                else:
                    row_v[pl.ds(k * 16, 16)] = (
                        row_v[pl.ds(k * 16, 16)] + pc * ws)
                    row_v[pl.ds(_FC + k * 16, 16)] = (
                        row_v[pl.ds(_FC + k * 16, 16)] + pd * wg)
        pltpu.sync_copy(row_v, out_hbm.at[r])

    emit_row(wid)

    @pl.when(wid < 8)
    def _():
        emit_row(wid + 32)


def kernel(x, W1, B1, W2, B2, Wsem, bsem, Wgen, bgen):
    B, C, W, H = x.shape
    xi = x.reshape(B * C, W, H)

    idx = jnp.arange(_IMG, dtype=jnp.int32)
    m = (idx[:, None] * idx[None, :]) % _IMG
    theta = (2.0 * np.pi / _IMG) * m.astype(jnp.float32)
    cm = jnp.cos(theta)
    sm = jnp.sin(theta)

    stats = pl.pallas_call(
        _dft_stats_kernel,
        grid=(B * C // 2,),
        in_specs=[
            pl.BlockSpec((2, _IMG, _IMG), lambda i: (i, 0, 0)),
            pl.BlockSpec((_IMG, _IMG), lambda i: (0, 0)),
            pl.BlockSpec((_IMG, _IMG), lambda i: (0, 0)),
        ],
        out_specs=pl.BlockSpec((2, 1, 128), lambda i: (i, 0, 0)),
        out_shape=jax.ShapeDtypeStruct((B * C, 1, 128), jnp.float32),
    )(xi, cm, sm)
    stats_s = stats.reshape(B * C, 128)[:, :16].reshape(-1)

    rows = []
    for Wm, Bm in ((W1, B1), (W2, B2)):
        for a in range(2):
            for arr in (Wm[a, 1], Bm[a, 1], Wm[a, 2], Bm[a, 2]):
                rows.append(jnp.pad(arr, (0, 16 - _NP)))
    wconst = jnp.concatenate(
        [jnp.stack(rows).reshape(-1), Wsem.reshape(-1), bsem,
         Wgen.reshape(-1), bgen])

    mesh = plsc.VectorSubcoreMesh(core_axis_name="c", subcore_axis_name="s")
    epilogue = pl.kernel(
        _epilogue_sc_kernel,
        mesh=mesh,
        out_type=jax.ShapeDtypeStruct((B * _NP, 2 * _FC), jnp.float32),
        scratch_types=[
            pltpu.VMEM((12 * 16,), jnp.float32),
            pltpu.VMEM((2304,), jnp.float32),
            pltpu.VMEM((24 * 16,), jnp.float32),
            pltpu.VMEM((2 * _FC,), jnp.float32),
        ],
    )
    return epilogue(stats_s, wconst)


# speedup vs baseline: 1.3575x; 1.1643x over previous
import functools
import jax
import jax.numpy as jnp
import numpy as np
from jax import lax
from jax.experimental import pallas as pl
from jax.experimental.pallas import tpu as pltpu
from jax.experimental.pallas import tpu_sc as plsc

_NP = 10
_IMG = 256
_HALF = _IMG // 2 + 1
_NBINS = _IMG * _HALF
_FC = 256


def _split(a):
    ah = a.astype(jnp.bfloat16)
    al = (a - ah.astype(jnp.float32)).astype(jnp.bfloat16)
    return ah, al


def _dot3(ah, al, bh, bl):
    d = lambda u, v: jax.lax.dot(u, v, preferred_element_type=jnp.float32)
    return d(ah, bh) + d(ah, bl) + d(al, bh)


def _dft_stats_kernel(x_ref, cmh_h_ref, cmh_l_ref, smh_h_ref, smh_l_ref,
                      cm_h_ref, cm_l_ref, sm_h_ref, sm_l_ref, out_ref):
    cmh = (cmh_h_ref[...], cmh_l_ref[...])
    smh = (smh_h_ref[...], smh_l_ref[...])
    cm = (cm_h_ref[...], cm_l_ref[...])
    sm = (sm_h_ref[...], sm_l_ref[...])
    alt = jnp.where(
        jax.lax.broadcasted_iota(jnp.int32, (_IMG, _IMG), 1) % 2 == 0,
        1.0, -1.0)
    lane = jax.lax.broadcasted_iota(jnp.int32, (1, 128), 1)
    lane2 = jax.lax.broadcasted_iota(jnp.int32, (_IMG, 2), 1)

    gs, s1s, s2s, dcs = [], [], [], []
    for j in range(2):
        x = x_ref[j]
        xs = _split(x)
        p = _dot3(*xs, *cmh)
        q = _dot3(*xs, *smh)
        ps = _split(p)
        qs = _split(q)
        fre = _dot3(*cm, *ps) - _dot3(*sm, *qs)
        fim = -(_dot3(*cm, *qs) + _dot3(*sm, *ps))
        mag = jnp.sqrt(fre * fre + fim * fim)
        ang = jnp.arctan2(fim, fre)
        gs.append(jnp.sum(x * alt, axis=1, keepdims=True))
        s1s.append(jnp.sum(mag))
        s2s.append(jnp.sum(ang))
        dcs.append(jnp.sum(x))

    gcat = jnp.concatenate(gs, axis=1)
    gsp = _split(gcat)
    f128re = _dot3(*cm, *gsp)
    f128im = -_dot3(*sm, *gsp)
    mag128 = jnp.sqrt(f128re * f128re + f128im * f128im)
    ang128 = jnp.arctan2(f128im, f128re)
    for j in range(2):
        sel = (lane2 == j).astype(jnp.float32)
        s1 = s1s[j] + jnp.sum(mag128 * sel)
        s2 = s2s[j] + jnp.sum(ang128 * sel)
        out_ref[j] = jnp.where(
            lane == 0, s1,
            jnp.where(lane == 1, s2, jnp.where(lane == 2, dcs[j], 0.0)))


def _sigmoid_v(v):
    return 1.0 / (1.0 + jnp.exp(-v))


def _band_mask_v(featv, w1, b1, w2, b2):
    c1 = _sigmoid_v(featv * w1 + b1)
    c2 = _sigmoid_v(featv * w2 + b2)
    ind = jnp.where(c2 > c1, 1.0, 0.0)
    return jnp.where(c1 * ind < 1.0 / _IMG, 1.0, 0.0)


def _epilogue_sc_kernel(stats_hbm, wconst_hbm, out_hbm,
                        stats_v, wconst_v, pooled_v, row_v):
    info = plsc.get_sparse_core_info()
    nc = info.num_cores
    wid = lax.axis_index("s") * nc + lax.axis_index("c")

    pltpu.sync_copy(stats_hbm, stats_v)
    pltpu.sync_copy(wconst_hbm, wconst_v)
    _HD = 256

    inv = jnp.float32(1.0 / (_IMG * _IMG))
    zeros16 = jnp.zeros((16,), jnp.int32)

    dnums = lax.GatherDimensionNumbers(
        offset_dims=(), collapsed_slice_dims=(0,), start_index_map=(0,))

    def splat(v, i):
        return lax.gather(v, (zeros16 + i)[:, None], dnums, (1,),
                          mode=lax.GatherScatterMode.PROMISE_IN_BOUNDS)

    for bc in range(12):
        srow = stats_v[pl.ds(bc * 16, 16)]
        feat1 = splat(srow, 0) * (1.0 / _NBINS)
        feat2 = splat(srow, 1) * (1.0 / _NBINS)
        dcv = splat(srow, 2)
        masks = []
        for m, featv in ((0, feat1), (1, feat2)):
            mrow = lambda a, k: wconst_v[pl.ds(((m * 2 + a) * 4 + k) * 16, 16)]
            mx = _band_mask_v(featv, mrow(0, 0), mrow(0, 1),
                              mrow(0, 2), mrow(0, 3))
            my = _band_mask_v(featv, mrow(1, 0), mrow(1, 1),
                              mrow(1, 2), mrow(1, 3))
            masks.append(jnp.minimum(mx + my, 1.0))
        mask1, mask2 = masks
        amp = jnp.abs(dcv) * inv
        negf = jnp.where(dcv < 0.0, 1.0, 0.0)
        cos_d = 1.0 - 2.0 * negf * mask2
        cos_c = 1.0 - 2.0 * negf * (1.0 - mask2)
        pooled_v[pl.ds(bc * 16, 16)] = amp * mask1 * cos_d
        pooled_v[pl.ds((12 + bc) * 16, 16)] = amp * (1.0 - mask1) * cos_c

    def emit_row(r):
        b = r // _NP
        n = r % _NP
        for c in range(3):
            pdrow = pooled_v[pl.ds((b * 3 + c) * 16, 16)]
            pcrow = pooled_v[pl.ds((12 + b * 3 + c) * 16, 16)]
            pd = splat(pdrow, n)
            pc = splat(pcrow, n)
            for k in range(_FC // 16):
                ws = wconst_v[pl.ds(_HD + c * _FC + k * 16, 16)]
                wg = wconst_v[pl.ds(_HD + (4 + c) * _FC + k * 16, 16)]
                if c == 0:
                    bs = wconst_v[pl.ds(_HD + 3 * _FC + k * 16, 16)]
                    bg = wconst_v[pl.ds(_HD + 7 * _FC + k * 16, 16)]
                    row_v[pl.ds(k * 16, 16)] = bs + pc * ws
                    row_v[pl.ds(_FC + k * 16, 16)] = bg + pd * wg
                else:
                    row_v[pl.ds(k * 16, 16)] = (
                        row_v[pl.ds(k * 16, 16)] + pc * ws)
                    row_v[pl.ds(_FC + k * 16, 16)] = (
                        row_v[pl.ds(_FC + k * 16, 16)] + pd * wg)
        pltpu.sync_copy(row_v, out_hbm.at[r])

    emit_row(wid)

    @pl.when(wid < 8)
    def _():
        emit_row(wid + 32)


def kernel(x, W1, B1, W2, B2, Wsem, bsem, Wgen, bgen):
    B, C, W, H = x.shape
    xi = x.reshape(B * C, W, H)

    idx = jnp.arange(_IMG, dtype=jnp.int32)
    m = (idx[:, None] * idx[None, :]) % _IMG
    theta = (2.0 * np.pi / _IMG) * m.astype(jnp.float32)
    cm = jnp.cos(theta)
    sm = jnp.sin(theta)
    cm_h, cm_l = _split(cm)
    sm_h, sm_l = _split(sm)

    full = lambda: pl.BlockSpec((_IMG, _IMG), lambda i: (0, 0))
    half = lambda: pl.BlockSpec((_IMG, 128), lambda i: (0, 0))
    stats = pl.pallas_call(
        _dft_stats_kernel,
        grid=(B * C // 2,),
        in_specs=[
            pl.BlockSpec((2, _IMG, _IMG), lambda i: (i, 0, 0)),
            half(), half(), half(), half(),
            full(), full(), full(), full(),
        ],
        out_specs=pl.BlockSpec((2, 1, 128), lambda i: (i, 0, 0)),
        out_shape=jax.ShapeDtypeStruct((B * C, 1, 128), jnp.float32),
    )(xi, cm_h[:, :128], cm_l[:, :128], sm_h[:, :128], sm_l[:, :128],
      cm_h, cm_l, sm_h, sm_l)
    stats_s = stats.reshape(B * C, 128)[:, :16].reshape(-1)

    rows = []
    for Wm, Bm in ((W1, B1), (W2, B2)):
        for a in range(2):
            for arr in (Wm[a, 1], Bm[a, 1], Wm[a, 2], Bm[a, 2]):
                rows.append(jnp.pad(arr, (0, 16 - _NP)))
    wconst = jnp.concatenate(
        [jnp.stack(rows).reshape(-1), Wsem.reshape(-1), bsem,
         Wgen.reshape(-1), bgen])

    mesh = plsc.VectorSubcoreMesh(core_axis_name="c", subcore_axis_name="s")
    epilogue = pl.kernel(
        _epilogue_sc_kernel,
        mesh=mesh,
        out_type=jax.ShapeDtypeStruct((B * _NP, 2 * _FC), jnp.float32),
        scratch_types=[
            pltpu.VMEM((12 * 16,), jnp.float32),
            pltpu.VMEM((2304,), jnp.float32),
            pltpu.VMEM((24 * 16,), jnp.float32),
            pltpu.VMEM((2 * _FC,), jnp.float32),
        ],
    )
    return epilogue(stats_s, wconst)


# speedup vs baseline: 1.3835x; 1.0191x over previous
import functools
import jax
import jax.numpy as jnp
import numpy as np
from jax import lax
from jax.experimental import pallas as pl
from jax.experimental.pallas import tpu as pltpu
from jax.experimental.pallas import tpu_sc as plsc

_NP = 10
_IMG = 256
_HALF = _IMG // 2 + 1
_NBINS = _IMG * _HALF
_FC = 256


def _split(a):
    ah = a.astype(jnp.bfloat16)
    al = (a - ah.astype(jnp.float32)).astype(jnp.bfloat16)
    return ah, al


def _dot3(ah, al, bh, bl):
    d = lambda u, v: jax.lax.dot(u, v, preferred_element_type=jnp.float32)
    return d(ah, bh) + d(ah, bl) + d(al, bh)


def _dft_stats_kernel(x_ref, cmh_h_ref, cmh_l_ref, smh_h_ref, smh_l_ref,
                      cm_h_ref, cm_l_ref, sm_h_ref, sm_l_ref, out_ref):
    cmh = (cmh_h_ref[...], cmh_l_ref[...])
    smh = (smh_h_ref[...], smh_l_ref[...])
    cm = (cm_h_ref[...], cm_l_ref[...])
    sm = (sm_h_ref[...], sm_l_ref[...])
    alt = jnp.where(
        jax.lax.broadcasted_iota(jnp.int32, (_IMG, _IMG), 1) % 2 == 0,
        1.0, -1.0)
    lane = jax.lax.broadcasted_iota(jnp.int32, (1, 16), 1)
    lane2 = jax.lax.broadcasted_iota(jnp.int32, (_IMG, 2), 1)

    gs, s1s, s2s, dcs = [], [], [], []
    for j in range(2):
        x = x_ref[j]
        xs = _split(x)
        p = _dot3(*xs, *cmh)
        q = _dot3(*xs, *smh)
        ps = _split(p)
        qs = _split(q)
        fre = _dot3(*cm, *ps) - _dot3(*sm, *qs)
        fim = -(_dot3(*cm, *qs) + _dot3(*sm, *ps))
        mag = jnp.sqrt(fre * fre + fim * fim)
        ang = jnp.arctan2(fim, fre)
        gs.append(jnp.sum(x * alt, axis=1, keepdims=True))
        s1s.append(jnp.sum(mag))
        s2s.append(jnp.sum(ang))
        dcs.append(jnp.sum(x))

    gcat = jnp.concatenate(gs, axis=1)
    gsp = _split(gcat)
    f128re = _dot3(*cm, *gsp)
    f128im = -_dot3(*sm, *gsp)
    mag128 = jnp.sqrt(f128re * f128re + f128im * f128im)
    ang128 = jnp.arctan2(f128im, f128re)
    for j in range(2):
        sel = (lane2 == j).astype(jnp.float32)
        s1 = s1s[j] + jnp.sum(mag128 * sel)
        s2 = s2s[j] + jnp.sum(ang128 * sel)
        out_ref[j] = jnp.where(
            lane == 0, s1,
            jnp.where(lane == 1, s2, jnp.where(lane == 2, dcs[j], 0.0)))


def _sigmoid_v(v):
    return 1.0 / (1.0 + jnp.exp(-v))


def _band_mask_v(featv, w1, b1, w2, b2):
    c1 = _sigmoid_v(featv * w1 + b1)
    c2 = _sigmoid_v(featv * w2 + b2)
    ind = jnp.where(c2 > c1, 1.0, 0.0)
    return jnp.where(c1 * ind < 1.0 / _IMG, 1.0, 0.0)


def _epilogue_sc_kernel(stats_hbm, wconst_hbm, out_hbm,
                        stats_v, wconst_v, pooled_v, row_v, row2_v,
                        sem_in, sem_r1, sem_r2):
    info = plsc.get_sparse_core_info()
    nc = info.num_cores
    wid = lax.axis_index("s") * nc + lax.axis_index("c")

    cp1 = pltpu.async_copy(stats_hbm, stats_v, sem_in)
    cp2 = pltpu.async_copy(wconst_hbm, wconst_v, sem_in)
    cp1.wait()
    cp2.wait()
    _HD = 256

    inv = jnp.float32(1.0 / (_IMG * _IMG))
    zeros16 = jnp.zeros((16,), jnp.int32)

    dnums = lax.GatherDimensionNumbers(
        offset_dims=(), collapsed_slice_dims=(0,), start_index_map=(0,))

    def splat(v, i):
        return lax.gather(v, (zeros16 + i)[:, None], dnums, (1,),
                          mode=lax.GatherScatterMode.PROMISE_IN_BOUNDS)

    for bc in range(12):
        srow = stats_v[pl.ds(bc * 16, 16)]
        feat1 = splat(srow, 0) * (1.0 / _NBINS)
        feat2 = splat(srow, 1) * (1.0 / _NBINS)
        dcv = splat(srow, 2)
        masks = []
        for m, featv in ((0, feat1), (1, feat2)):
            mrow = lambda a, k: wconst_v[pl.ds(((m * 2 + a) * 4 + k) * 16, 16)]
            mx = _band_mask_v(featv, mrow(0, 0), mrow(0, 1),
                              mrow(0, 2), mrow(0, 3))
            my = _band_mask_v(featv, mrow(1, 0), mrow(1, 1),
                              mrow(1, 2), mrow(1, 3))
            masks.append(jnp.minimum(mx + my, 1.0))
        mask1, mask2 = masks
        amp = jnp.abs(dcv) * inv
        negf = jnp.where(dcv < 0.0, 1.0, 0.0)
        cos_d = 1.0 - 2.0 * negf * mask2
        cos_c = 1.0 - 2.0 * negf * (1.0 - mask2)
        pooled_v[pl.ds(bc * 16, 16)] = amp * mask1 * cos_d
        pooled_v[pl.ds((12 + bc) * 16, 16)] = amp * (1.0 - mask1) * cos_c

    def emit_row(r, buf, sem):
        b = r // _NP
        n = r % _NP
        for c in range(3):
            pdrow = pooled_v[pl.ds((b * 3 + c) * 16, 16)]
            pcrow = pooled_v[pl.ds((12 + b * 3 + c) * 16, 16)]
            pd = splat(pdrow, n)
            pc = splat(pcrow, n)
            for k in range(_FC // 16):
                ws = wconst_v[pl.ds(_HD + c * _FC + k * 16, 16)]
                wg = wconst_v[pl.ds(_HD + (4 + c) * _FC + k * 16, 16)]
                if c == 0:
                    bs = wconst_v[pl.ds(_HD + 3 * _FC + k * 16, 16)]
                    bg = wconst_v[pl.ds(_HD + 7 * _FC + k * 16, 16)]
                    buf[pl.ds(k * 16, 16)] = bs + pc * ws
                    buf[pl.ds(_FC + k * 16, 16)] = bg + pd * wg
                else:
                    buf[pl.ds(k * 16, 16)] = (
                        buf[pl.ds(k * 16, 16)] + pc * ws)
                    buf[pl.ds(_FC + k * 16, 16)] = (
                        buf[pl.ds(_FC + k * 16, 16)] + pd * wg)
        return pltpu.async_copy(buf, out_hbm.at[r], sem)

    d1 = emit_row(wid, row_v, sem_r1)

    @pl.when(wid < 8)
    def _():
        emit_row(wid + 32, row2_v, sem_r2).wait()

    d1.wait()


def kernel(x, W1, B1, W2, B2, Wsem, bsem, Wgen, bgen):
    B, C, W, H = x.shape
    xi = x.reshape(B * C, W, H)

    idx = jnp.arange(_IMG, dtype=jnp.int32)
    m = (idx[:, None] * idx[None, :]) % _IMG
    theta = (2.0 * np.pi / _IMG) * m.astype(jnp.float32)
    cm = jnp.cos(theta)
    sm = jnp.sin(theta)
    cm_h, cm_l = _split(cm)
    sm_h, sm_l = _split(sm)

    full = lambda: pl.BlockSpec((_IMG, _IMG), lambda i: (0, 0))
    half = lambda: pl.BlockSpec((_IMG, 128), lambda i: (0, 0))
    stats = pl.pallas_call(
        _dft_stats_kernel,
        grid=(B * C // 2,),
        in_specs=[
            pl.BlockSpec((2, _IMG, _IMG), lambda i: (i, 0, 0)),
            half(), half(), half(), half(),
            full(), full(), full(), full(),
        ],
        out_specs=pl.BlockSpec((2, 1, 16), lambda i: (i, 0, 0)),
        out_shape=jax.ShapeDtypeStruct((B * C, 1, 16), jnp.float32),
    )(xi, cm_h[:, :128], cm_l[:, :128], sm_h[:, :128], sm_l[:, :128],
      cm_h, cm_l, sm_h, sm_l)
    stats_s = stats.reshape(-1)

    rows = []
    for Wm, Bm in ((W1, B1), (W2, B2)):
        for a in range(2):
            for arr in (Wm[a, 1], Bm[a, 1], Wm[a, 2], Bm[a, 2]):
                rows.append(jnp.pad(arr, (0, 16 - _NP)))
    wconst = jnp.concatenate(
        [jnp.stack(rows).reshape(-1), Wsem.reshape(-1), bsem,
         Wgen.reshape(-1), bgen])

    mesh = plsc.VectorSubcoreMesh(core_axis_name="c", subcore_axis_name="s")
    epilogue = pl.kernel(
        _epilogue_sc_kernel,
        mesh=mesh,
        out_type=jax.ShapeDtypeStruct((B * _NP, 2 * _FC), jnp.float32),
        scratch_types=[
            pltpu.VMEM((12 * 16,), jnp.float32),
            pltpu.VMEM((2304,), jnp.float32),
            pltpu.VMEM((24 * 16,), jnp.float32),
            pltpu.VMEM((2 * _FC,), jnp.float32),
            pltpu.VMEM((2 * _FC,), jnp.float32),
            pltpu.SemaphoreType.DMA,
            pltpu.SemaphoreType.DMA,
            pltpu.SemaphoreType.DMA,
        ],
    )
    return epilogue(stats_s, wconst)


# speedup vs baseline: 1.3932x; 1.0071x over previous
import functools
import jax
import jax.numpy as jnp
import numpy as np
from jax import lax
from jax.experimental import pallas as pl
from jax.experimental.pallas import tpu as pltpu
from jax.experimental.pallas import tpu_sc as plsc

_NP = 10
_IMG = 256
_HALF = _IMG // 2 + 1
_NBINS = _IMG * _HALF
_FC = 256


def _split(a):
    ah = a.astype(jnp.bfloat16)
    al = (a - ah.astype(jnp.float32)).astype(jnp.bfloat16)
    return ah, al


def _dot3(ah, al, bh, bl):
    d = lambda u, v: jax.lax.dot(u, v, preferred_element_type=jnp.float32)
    return d(ah, bh) + d(ah, bl) + d(al, bh)


def _dft_stats_kernel(x_ref, cmh_h_ref, cmh_l_ref, smh_h_ref, smh_l_ref,
                      cm_h_ref, cm_l_ref, sm_h_ref, sm_l_ref, out_ref):
    cmh = (cmh_h_ref[...], cmh_l_ref[...])
    smh = (smh_h_ref[...], smh_l_ref[...])
    cm = (cm_h_ref[...], cm_l_ref[...])
    sm = (sm_h_ref[...], sm_l_ref[...])
    alt = jnp.where(
        jax.lax.broadcasted_iota(jnp.int32, (_IMG, _IMG), 1) % 2 == 0,
        1.0, -1.0)
    lane = jax.lax.broadcasted_iota(jnp.int32, (3, 16), 1)
    rowi = jax.lax.broadcasted_iota(jnp.int32, (3, 16), 0)
    lane2 = jax.lax.broadcasted_iota(jnp.int32, (_IMG, 3), 1)

    gs, s1s, s2s, dcs = [], [], [], []
    for j in range(3):
        x = x_ref[0, j]
        xs = _split(x)
        p = _dot3(*xs, *cmh)
        q = _dot3(*xs, *smh)
        ps = _split(p)
        qs = _split(q)
        fre = _dot3(*cm, *ps) - _dot3(*sm, *qs)
        fim = -(_dot3(*cm, *qs) + _dot3(*sm, *ps))
        mag = jnp.sqrt(fre * fre + fim * fim)
        ang = jnp.arctan2(fim, fre)
        gs.append(jnp.sum(x * alt, axis=1, keepdims=True))
        s1s.append(jnp.sum(mag))
        s2s.append(jnp.sum(ang))
        dcs.append(jnp.sum(x))

    gcat = jnp.concatenate(gs, axis=1)
    gsp = _split(gcat)
    f128re = _dot3(*cm, *gsp)
    f128im = -_dot3(*sm, *gsp)
    mag128 = jnp.sqrt(f128re * f128re + f128im * f128im)
    ang128 = jnp.arctan2(f128im, f128re)
    rows = []
    for j in range(3):
        sel = (lane2 == j).astype(jnp.float32)
        s1 = s1s[j] + jnp.sum(mag128 * sel)
        s2 = s2s[j] + jnp.sum(ang128 * sel)
        rows.append(jnp.where(
            lane == 0, s1,
            jnp.where(lane == 1, s2, jnp.where(lane == 2, dcs[j], 0.0))))
    out_ref[0] = jnp.where(rowi == 0, rows[0],
                           jnp.where(rowi == 1, rows[1], rows[2]))


def _sigmoid_v(v):
    return 1.0 / (1.0 + jnp.exp(-v))


def _band_mask_v(featv, w1, b1, w2, b2):
    c1 = _sigmoid_v(featv * w1 + b1)
    c2 = _sigmoid_v(featv * w2 + b2)
    ind = jnp.where(c2 > c1, 1.0, 0.0)
    return jnp.where(c1 * ind < 1.0 / _IMG, 1.0, 0.0)


def _epilogue_sc_kernel(stats_hbm, wconst_hbm, out_hbm,
                        stats_v, wconst_v, pooled_v, row_v, row2_v,
                        sem_in, sem_r1, sem_r2):
    info = plsc.get_sparse_core_info()
    nc = info.num_cores
    wid = lax.axis_index("s") * nc + lax.axis_index("c")

    cp1 = pltpu.async_copy(stats_hbm, stats_v, sem_in)
    cp2 = pltpu.async_copy(wconst_hbm, wconst_v, sem_in)
    cp1.wait()
    cp2.wait()
    _HD = 256

    inv = jnp.float32(1.0 / (_IMG * _IMG))
    zeros16 = jnp.zeros((16,), jnp.int32)

    dnums = lax.GatherDimensionNumbers(
        offset_dims=(), collapsed_slice_dims=(0,), start_index_map=(0,))

    def splat(v, i):
        return lax.gather(v, (zeros16 + i)[:, None], dnums, (1,),
                          mode=lax.GatherScatterMode.PROMISE_IN_BOUNDS)

    for bc in range(12):
        srow = stats_v[pl.ds(bc * 16, 16)]
        feat1 = splat(srow, 0) * (1.0 / _NBINS)
        feat2 = splat(srow, 1) * (1.0 / _NBINS)
        dcv = splat(srow, 2)
        masks = []
        for m, featv in ((0, feat1), (1, feat2)):
            mrow = lambda a, k: wconst_v[pl.ds(((m * 2 + a) * 4 + k) * 16, 16)]
            mx = _band_mask_v(featv, mrow(0, 0), mrow(0, 1),
                              mrow(0, 2), mrow(0, 3))
            my = _band_mask_v(featv, mrow(1, 0), mrow(1, 1),
                              mrow(1, 2), mrow(1, 3))
            masks.append(jnp.minimum(mx + my, 1.0))
        mask1, mask2 = masks
        amp = jnp.abs(dcv) * inv
        negf = jnp.where(dcv < 0.0, 1.0, 0.0)
        cos_d = 1.0 - 2.0 * negf * mask2
        cos_c = 1.0 - 2.0 * negf * (1.0 - mask2)
        pooled_v[pl.ds(bc * 16, 16)] = amp * mask1 * cos_d
        pooled_v[pl.ds((12 + bc) * 16, 16)] = amp * (1.0 - mask1) * cos_c

    def emit_row(r, buf, sem):
        b = r // _NP
        n = r % _NP
        for c in range(3):
            pdrow = pooled_v[pl.ds((b * 3 + c) * 16, 16)]
            pcrow = pooled_v[pl.ds((12 + b * 3 + c) * 16, 16)]
            pd = splat(pdrow, n)
            pc = splat(pcrow, n)
            for k in range(_FC // 16):
                ws = wconst_v[pl.ds(_HD + c * _FC + k * 16, 16)]
                wg = wconst_v[pl.ds(_HD + (4 + c) * _FC + k * 16, 16)]
                if c == 0:
                    bs = wconst_v[pl.ds(_HD + 3 * _FC + k * 16, 16)]
                    bg = wconst_v[pl.ds(_HD + 7 * _FC + k * 16, 16)]
                    buf[pl.ds(k * 16, 16)] = bs + pc * ws
                    buf[pl.ds(_FC + k * 16, 16)] = bg + pd * wg
                else:
                    buf[pl.ds(k * 16, 16)] = (
                        buf[pl.ds(k * 16, 16)] + pc * ws)
                    buf[pl.ds(_FC + k * 16, 16)] = (
                        buf[pl.ds(_FC + k * 16, 16)] + pd * wg)
        return pltpu.async_copy(buf, out_hbm.at[r], sem)

    d1 = emit_row(wid, row_v, sem_r1)

    @pl.when(wid < 8)
    def _():
        emit_row(wid + 32, row2_v, sem_r2).wait()

    d1.wait()


def kernel(x, W1, B1, W2, B2, Wsem, bsem, Wgen, bgen):
    B, C, W, H = x.shape

    idx = jnp.arange(_IMG, dtype=jnp.int32)
    m = (idx[:, None] * idx[None, :]) % _IMG
    theta = (2.0 * np.pi / _IMG) * m.astype(jnp.float32)
    cm = jnp.cos(theta)
    sm = jnp.sin(theta)
    cm_h, cm_l = _split(cm)
    sm_h, sm_l = _split(sm)

    full = lambda: pl.BlockSpec((_IMG, _IMG), lambda i: (0, 0))
    half = lambda: pl.BlockSpec((_IMG, 128), lambda i: (0, 0))
    stats = pl.pallas_call(
        _dft_stats_kernel,
        grid=(B,),
        in_specs=[
            pl.BlockSpec((1, C, _IMG, _IMG), lambda i: (i, 0, 0, 0)),
            half(), half(), half(), half(),
            full(), full(), full(), full(),
        ],
        out_specs=pl.BlockSpec((1, 3, 16), lambda i: (i, 0, 0)),
        out_shape=jax.ShapeDtypeStruct((B, 3, 16), jnp.float32),
    )(x, cm_h[:, :128], cm_l[:, :128], sm_h[:, :128], sm_l[:, :128],
      cm_h, cm_l, sm_h, sm_l)
    stats_s = stats.reshape(-1)

    rows = []
    for Wm, Bm in ((W1, B1), (W2, B2)):
        for a in range(2):
            for arr in (Wm[a, 1], Bm[a, 1], Wm[a, 2], Bm[a, 2]):
                rows.append(jnp.pad(arr, (0, 16 - _NP)))
    wconst = jnp.concatenate(
        [jnp.stack(rows).reshape(-1), Wsem.reshape(-1), bsem,
         Wgen.reshape(-1), bgen])

    mesh = plsc.VectorSubcoreMesh(core_axis_name="c", subcore_axis_name="s")
    epilogue = pl.kernel(
        _epilogue_sc_kernel,
        mesh=mesh,
        out_type=jax.ShapeDtypeStruct((B * _NP, 2 * _FC), jnp.float32),
        scratch_types=[
            pltpu.VMEM((12 * 16,), jnp.float32),
            pltpu.VMEM((2304,), jnp.float32),
            pltpu.VMEM((24 * 16,), jnp.float32),
            pltpu.VMEM((2 * _FC,), jnp.float32),
            pltpu.VMEM((2 * _FC,), jnp.float32),
            pltpu.SemaphoreType.DMA,
            pltpu.SemaphoreType.DMA,
            pltpu.SemaphoreType.DMA,
        ],
    )
    return epilogue(stats_s, wconst)


# speedup vs baseline: 1.4253x; 1.0230x over previous
import functools
import jax
import jax.numpy as jnp
import numpy as np
from jax import lax
from jax.experimental import pallas as pl
from jax.experimental.pallas import tpu as pltpu
from jax.experimental.pallas import tpu_sc as plsc

_NP = 10
_IMG = 256
_HALF = _IMG // 2 + 1
_NBINS = _IMG * _HALF
_FC = 256


def _split(a):
    ah = a.astype(jnp.bfloat16)
    al = (a - ah.astype(jnp.float32)).astype(jnp.bfloat16)
    return ah, al


def _dot3(ah, al, bh, bl):
    d = lambda u, v: jax.lax.dot(u, v, preferred_element_type=jnp.float32)
    return d(ah, bh) + d(ah, bl) + d(al, bh)


def _dft_stats_kernel(x_ref, cmh_h_ref, cmh_l_ref, smh_h_ref, smh_l_ref,
                      cm_h_ref, cm_l_ref, sm_h_ref, sm_l_ref, out_ref):
    cmh = (cmh_h_ref[...], cmh_l_ref[...])
    smh = (smh_h_ref[...], smh_l_ref[...])
    cm = (cm_h_ref[...], cm_l_ref[...])
    sm = (sm_h_ref[...], sm_l_ref[...])
    alt = jnp.where(
        jax.lax.broadcasted_iota(jnp.int32, (_IMG, _IMG), 1) % 2 == 0,
        1.0, -1.0)
    lane = jax.lax.broadcasted_iota(jnp.int32, (3, 16), 1)
    rowi = jax.lax.broadcasted_iota(jnp.int32, (3, 16), 0)
    lane2 = jax.lax.broadcasted_iota(jnp.int32, (_IMG, 3), 1)

    gs, s1s, s2s, dcs = [], [], [], []
    for j in range(3):
        x = x_ref[0, j]
        xs = _split(x)
        p = _dot3(*xs, *cmh)
        q = _dot3(*xs, *smh)
        ps = _split(p)
        qs = _split(q)
        fre = _dot3(*cm, *ps) - _dot3(*sm, *qs)
        fim = -(_dot3(*cm, *qs) + _dot3(*sm, *ps))
        mag = jnp.sqrt(fre * fre + fim * fim)
        ang = jnp.arctan2(fim, fre)
        gs.append(jnp.sum(x * alt, axis=1, keepdims=True))
        s1s.append(jnp.sum(mag))
        s2s.append(jnp.sum(ang))
        dcs.append(jnp.sum(x))

    gcat = jnp.concatenate(gs, axis=1)
    gsp = _split(gcat)
    f128re = _dot3(*cm, *gsp)
    f128im = -_dot3(*sm, *gsp)
    mag128 = jnp.sqrt(f128re * f128re + f128im * f128im)
    ang128 = jnp.arctan2(f128im, f128re)
    rows = []
    for j in range(3):
        sel = (lane2 == j).astype(jnp.float32)
        s1 = s1s[j] + jnp.sum(mag128 * sel)
        s2 = s2s[j] + jnp.sum(ang128 * sel)
        rows.append(jnp.where(
            lane == 0, s1,
            jnp.where(lane == 1, s2, jnp.where(lane == 2, dcs[j], 0.0))))
    out_ref[0] = jnp.where(rowi == 0, rows[0],
                           jnp.where(rowi == 1, rows[1], rows[2]))


def _sigmoid_v(v):
    return 1.0 / (1.0 + jnp.exp(-v))


def _band_mask_v(featv, w1, b1, w2, b2):
    c1 = _sigmoid_v(featv * w1 + b1)
    c2 = _sigmoid_v(featv * w2 + b2)
    ind = jnp.where(c2 > c1, 1.0, 0.0)
    return jnp.where(c1 * ind < 1.0 / _IMG, 1.0, 0.0)


def _epilogue_sc_kernel(stats_hbm, wconst_hbm, out_hbm,
                        stats_v, wconst_v, row_v, row2_v,
                        sem_in, sem_r1, sem_r2):
    info = plsc.get_sparse_core_info()
    nc = info.num_cores
    wid = lax.axis_index("s") * nc + lax.axis_index("c")

    cp1 = pltpu.async_copy(stats_hbm, stats_v, sem_in)
    cp2 = pltpu.async_copy(wconst_hbm, wconst_v, sem_in)
    cp1.wait()
    cp2.wait()
    _HD = 256

    inv = jnp.float32(1.0 / (_IMG * _IMG))
    zeros16 = jnp.zeros((16,), jnp.int32)

    dnums = lax.GatherDimensionNumbers(
        offset_dims=(), collapsed_slice_dims=(0,), start_index_map=(0,))

    def splat(v, i):
        return lax.gather(v, (zeros16 + i)[:, None], dnums, (1,),
                          mode=lax.GatherScatterMode.PROMISE_IN_BOUNDS)

    def pooled(bc):
        srow = stats_v[pl.ds(bc * 16, 16)]
        feat1 = splat(srow, 0) * (1.0 / _NBINS)
        feat2 = splat(srow, 1) * (1.0 / _NBINS)
        dcv = splat(srow, 2)
        masks = []
        for m, featv in ((0, feat1), (1, feat2)):
            mrow = lambda a, k: wconst_v[pl.ds(((m * 2 + a) * 4 + k) * 16, 16)]
            mx = _band_mask_v(featv, mrow(0, 0), mrow(0, 1),
                              mrow(0, 2), mrow(0, 3))
            my = _band_mask_v(featv, mrow(1, 0), mrow(1, 1),
                              mrow(1, 2), mrow(1, 3))
            masks.append(jnp.minimum(mx + my, 1.0))
        mask1, mask2 = masks
        amp = jnp.abs(dcv) * inv
        negf = jnp.where(dcv < 0.0, 1.0, 0.0)
        cos_d = 1.0 - 2.0 * negf * mask2
        cos_c = 1.0 - 2.0 * negf * (1.0 - mask2)
        return amp * mask1 * cos_d, amp * (1.0 - mask1) * cos_c

    def emit_row(r, buf, sem):
        b = r // _NP
        n = r % _NP
        for c in range(3):
            pdrow, pcrow = pooled(b * 3 + c)
            pd = splat(pdrow, n)
            pc = splat(pcrow, n)
            for k in range(_FC // 16):
                ws = wconst_v[pl.ds(_HD + c * _FC + k * 16, 16)]
                wg = wconst_v[pl.ds(_HD + (4 + c) * _FC + k * 16, 16)]
                if c == 0:
                    bs = wconst_v[pl.ds(_HD + 3 * _FC + k * 16, 16)]
                    bg = wconst_v[pl.ds(_HD + 7 * _FC + k * 16, 16)]
                    buf[pl.ds(k * 16, 16)] = bs + pc * ws
                    buf[pl.ds(_FC + k * 16, 16)] = bg + pd * wg
                else:
                    buf[pl.ds(k * 16, 16)] = (
                        buf[pl.ds(k * 16, 16)] + pc * ws)
                    buf[pl.ds(_FC + k * 16, 16)] = (
                        buf[pl.ds(_FC + k * 16, 16)] + pd * wg)
        return pltpu.async_copy(buf, out_hbm.at[r], sem)

    d1 = emit_row(wid, row_v, sem_r1)

    @pl.when(wid < 8)
    def _():
        emit_row(wid + 32, row2_v, sem_r2).wait()

    d1.wait()


def kernel(x, W1, B1, W2, B2, Wsem, bsem, Wgen, bgen):
    B, C, W, H = x.shape

    idx = jnp.arange(_IMG, dtype=jnp.int32)
    m = (idx[:, None] * idx[None, :]) % _IMG
    theta = (2.0 * np.pi / _IMG) * m.astype(jnp.float32)
    cm = jnp.cos(theta)
    sm = jnp.sin(theta)
    cm_h, cm_l = _split(cm)
    sm_h, sm_l = _split(sm)

    full = lambda: pl.BlockSpec((_IMG, _IMG), lambda i: (0, 0))
    half = lambda: pl.BlockSpec((_IMG, 128), lambda i: (0, 0))
    stats = pl.pallas_call(
        _dft_stats_kernel,
        grid=(B,),
        in_specs=[
            pl.BlockSpec((1, C, _IMG, _IMG), lambda i: (i, 0, 0, 0)),
            half(), half(), half(), half(),
            full(), full(), full(), full(),
        ],
        out_specs=pl.BlockSpec((1, 3, 16), lambda i: (i, 0, 0)),
        out_shape=jax.ShapeDtypeStruct((B, 3, 16), jnp.float32),
    )(x, cm_h[:, :128], cm_l[:, :128], sm_h[:, :128], sm_l[:, :128],
      cm_h, cm_l, sm_h, sm_l)
    stats_s = stats.reshape(-1)

    rows = []
    for Wm, Bm in ((W1, B1), (W2, B2)):
        for a in range(2):
            for arr in (Wm[a, 1], Bm[a, 1], Wm[a, 2], Bm[a, 2]):
                rows.append(jnp.pad(arr, (0, 16 - _NP)))
    wconst = jnp.concatenate(
        [jnp.stack(rows).reshape(-1), Wsem.reshape(-1), bsem,
         Wgen.reshape(-1), bgen])

    mesh = plsc.VectorSubcoreMesh(core_axis_name="c", subcore_axis_name="s")
    epilogue = pl.kernel(
        _epilogue_sc_kernel,
        mesh=mesh,
        out_type=jax.ShapeDtypeStruct((B * _NP, 2 * _FC), jnp.float32),
        scratch_types=[
            pltpu.VMEM((12 * 16,), jnp.float32),
            pltpu.VMEM((2304,), jnp.float32),
            pltpu.VMEM((2 * _FC,), jnp.float32),
            pltpu.VMEM((2 * _FC,), jnp.float32),
            pltpu.SemaphoreType.DMA,
            pltpu.SemaphoreType.DMA,
            pltpu.SemaphoreType.DMA,
        ],
    )
    return epilogue(stats_s, wconst)


# speedup vs baseline: 1.4529x; 1.0194x over previous
import functools
import jax
import jax.numpy as jnp
import numpy as np
from jax import lax
from jax.experimental import pallas as pl
from jax.experimental.pallas import tpu as pltpu
from jax.experimental.pallas import tpu_sc as plsc

_NP = 10
_IMG = 256
_HALF = _IMG // 2 + 1
_NBINS = _IMG * _HALF
_FC = 256


def _split(a):
    ah = a.astype(jnp.bfloat16)
    al = (a - ah.astype(jnp.float32)).astype(jnp.bfloat16)
    return ah, al


def _dot3(ah, al, bh, bl):
    d = lambda u, v: jax.lax.dot(u, v, preferred_element_type=jnp.float32)
    return d(ah, bh) + d(ah, bl) + d(al, bh)


def _dft_stats_kernel(x_ref, cmh_h_ref, cmh_l_ref, smh_h_ref, smh_l_ref,
                      cm_h_ref, cm_l_ref, sm_h_ref, sm_l_ref, out_ref):
    cmh = (cmh_h_ref[...], cmh_l_ref[...])
    smh = (smh_h_ref[...], smh_l_ref[...])
    cm = (cm_h_ref[...], cm_l_ref[...])
    sm = (sm_h_ref[...], sm_l_ref[...])
    alt = jnp.where(
        jax.lax.broadcasted_iota(jnp.int32, (_IMG, _IMG), 1) % 2 == 0,
        1.0, -1.0)
    lane = jax.lax.broadcasted_iota(jnp.int32, (3, 16), 1)
    rowi = jax.lax.broadcasted_iota(jnp.int32, (3, 16), 0)
    lane2 = jax.lax.broadcasted_iota(jnp.int32, (_IMG, 6), 1)

    gs, s1s, s2s, dcs = [], [], [], []
    for j in range(6):
        x = x_ref[j // 3, j % 3]
        xs = _split(x)
        p = _dot3(*xs, *cmh)
        q = _dot3(*xs, *smh)
        ps = _split(p)
        qs = _split(q)
        fre = _dot3(*cm, *ps) - _dot3(*sm, *qs)
        fim = -(_dot3(*cm, *qs) + _dot3(*sm, *ps))
        mag = jnp.sqrt(fre * fre + fim * fim)
        ang = jnp.arctan2(fim, fre)
        gs.append(jnp.sum(x * alt, axis=1, keepdims=True))
        s1s.append(jnp.sum(mag))
        s2s.append(jnp.sum(ang))
        dcs.append(jnp.sum(x))

    gcat = jnp.concatenate(gs, axis=1)
    gsp = _split(gcat)
    f128re = _dot3(*cm, *gsp)
    f128im = -_dot3(*sm, *gsp)
    mag128 = jnp.sqrt(f128re * f128re + f128im * f128im)
    ang128 = jnp.arctan2(f128im, f128re)
    for i in range(2):
        rows = []
        for jj in range(3):
            j = i * 3 + jj
            sel = (lane2 == j).astype(jnp.float32)
            s1 = s1s[j] + jnp.sum(mag128 * sel)
            s2 = s2s[j] + jnp.sum(ang128 * sel)
            rows.append(jnp.where(
                lane == 0, s1,
                jnp.where(lane == 1, s2, jnp.where(lane == 2, dcs[j], 0.0))))
        out_ref[i] = jnp.where(rowi == 0, rows[0],
                               jnp.where(rowi == 1, rows[1], rows[2]))


def _sigmoid_v(v):
    return 1.0 / (1.0 + jnp.exp(-v))


def _band_mask_v(featv, w1, b1, w2, b2):
    c1 = _sigmoid_v(featv * w1 + b1)
    c2 = _sigmoid_v(featv * w2 + b2)
    ind = jnp.where(c2 > c1, 1.0, 0.0)
    return jnp.where(c1 * ind < 1.0 / _IMG, 1.0, 0.0)


def _epilogue_sc_kernel(stats_hbm, wconst_hbm, out_hbm,
                        stats_v, wconst_v, row_v, row2_v,
                        sem_in, sem_r1, sem_r2):
    info = plsc.get_sparse_core_info()
    nc = info.num_cores
    wid = lax.axis_index("s") * nc + lax.axis_index("c")

    cp1 = pltpu.async_copy(stats_hbm, stats_v, sem_in)
    cp2 = pltpu.async_copy(wconst_hbm, wconst_v, sem_in)
    cp1.wait()
    cp2.wait()
    _HD = 256

    inv = jnp.float32(1.0 / (_IMG * _IMG))
    zeros16 = jnp.zeros((16,), jnp.int32)

    dnums = lax.GatherDimensionNumbers(
        offset_dims=(), collapsed_slice_dims=(0,), start_index_map=(0,))

    def splat(v, i):
        return lax.gather(v, (zeros16 + i)[:, None], dnums, (1,),
                          mode=lax.GatherScatterMode.PROMISE_IN_BOUNDS)

    def pooled(bc):
        srow = stats_v[pl.ds(bc * 16, 16)]
        feat1 = splat(srow, 0) * (1.0 / _NBINS)
        feat2 = splat(srow, 1) * (1.0 / _NBINS)
        dcv = splat(srow, 2)
        masks = []
        for m, featv in ((0, feat1), (1, feat2)):
            mrow = lambda a, k: wconst_v[pl.ds(((m * 2 + a) * 4 + k) * 16, 16)]
            mx = _band_mask_v(featv, mrow(0, 0), mrow(0, 1),
                              mrow(0, 2), mrow(0, 3))
            my = _band_mask_v(featv, mrow(1, 0), mrow(1, 1),
                              mrow(1, 2), mrow(1, 3))
            masks.append(jnp.minimum(mx + my, 1.0))
        mask1, mask2 = masks
        amp = jnp.abs(dcv) * inv
        negf = jnp.where(dcv < 0.0, 1.0, 0.0)
        cos_d = 1.0 - 2.0 * negf * mask2
        cos_c = 1.0 - 2.0 * negf * (1.0 - mask2)
        return amp * mask1 * cos_d, amp * (1.0 - mask1) * cos_c

    def emit_row(r, buf, sem):
        b = r // _NP
        n = r % _NP
        for c in range(3):
            pdrow, pcrow = pooled(b * 3 + c)
            pd = splat(pdrow, n)
            pc = splat(pcrow, n)
            for k in range(_FC // 16):
                ws = wconst_v[pl.ds(_HD + c * _FC + k * 16, 16)]
                wg = wconst_v[pl.ds(_HD + (4 + c) * _FC + k * 16, 16)]
                if c == 0:
                    bs = wconst_v[pl.ds(_HD + 3 * _FC + k * 16, 16)]
                    bg = wconst_v[pl.ds(_HD + 7 * _FC + k * 16, 16)]
                    buf[pl.ds(k * 16, 16)] = bs + pc * ws
                    buf[pl.ds(_FC + k * 16, 16)] = bg + pd * wg
                else:
                    buf[pl.ds(k * 16, 16)] = (
                        buf[pl.ds(k * 16, 16)] + pc * ws)
                    buf[pl.ds(_FC + k * 16, 16)] = (
                        buf[pl.ds(_FC + k * 16, 16)] + pd * wg)
        return pltpu.async_copy(buf, out_hbm.at[r], sem)

    d1 = emit_row(wid, row_v, sem_r1)

    @pl.when(wid < 8)
    def _():
        emit_row(wid + 32, row2_v, sem_r2).wait()

    d1.wait()


def kernel(x, W1, B1, W2, B2, Wsem, bsem, Wgen, bgen):
    B, C, W, H = x.shape

    idx = jnp.arange(_IMG, dtype=jnp.int32)
    m = (idx[:, None] * idx[None, :]) % _IMG
    theta = (2.0 * np.pi / _IMG) * m.astype(jnp.float32)
    cm = jnp.cos(theta)
    sm = jnp.sin(theta)
    cm_h, cm_l = _split(cm)
    sm_h, sm_l = _split(sm)

    full = lambda: pl.BlockSpec((_IMG, _IMG), lambda i: (0, 0))
    half = lambda: pl.BlockSpec((_IMG, 128), lambda i: (0, 0))
    stats = pl.pallas_call(
        _dft_stats_kernel,
        grid=(B // 2,),
        in_specs=[
            pl.BlockSpec((2, C, _IMG, _IMG), lambda i: (i, 0, 0, 0)),
            half(), half(), half(), half(),
            full(), full(), full(), full(),
        ],
        out_specs=pl.BlockSpec((2, 3, 16), lambda i: (i, 0, 0)),
        out_shape=jax.ShapeDtypeStruct((B, 3, 16), jnp.float32),
    )(x, cm_h[:, :128], cm_l[:, :128], sm_h[:, :128], sm_l[:, :128],
      cm_h, cm_l, sm_h, sm_l)
    stats_s = stats.reshape(-1)

    rows = []
    for Wm, Bm in ((W1, B1), (W2, B2)):
        for a in range(2):
            for arr in (Wm[a, 1], Bm[a, 1], Wm[a, 2], Bm[a, 2]):
                rows.append(jnp.pad(arr, (0, 16 - _NP)))
    wconst = jnp.concatenate(
        [jnp.stack(rows).reshape(-1), Wsem.reshape(-1), bsem,
         Wgen.reshape(-1), bgen])

    mesh = plsc.VectorSubcoreMesh(core_axis_name="c", subcore_axis_name="s")
    epilogue = pl.kernel(
        _epilogue_sc_kernel,
        mesh=mesh,
        out_type=jax.ShapeDtypeStruct((B * _NP, 2 * _FC), jnp.float32),
        scratch_types=[
            pltpu.VMEM((12 * 16,), jnp.float32),
            pltpu.VMEM((2304,), jnp.float32),
            pltpu.VMEM((2 * _FC,), jnp.float32),
            pltpu.VMEM((2 * _FC,), jnp.float32),
            pltpu.SemaphoreType.DMA,
            pltpu.SemaphoreType.DMA,
            pltpu.SemaphoreType.DMA,
        ],
    )
    return epilogue(stats_s, wconst)


# speedup vs baseline: 1.4602x; 1.0050x over previous
import jax
import jax.numpy as jnp
import numpy as np
from jax import lax
from jax.experimental import pallas as pl
from jax.experimental.pallas import tpu as pltpu
from jax.experimental.pallas import tpu_sc as plsc

_NP = 10
_IMG = 256
_HALF = _IMG // 2 + 1
_NBINS = _IMG * _HALF
_FC = 256


def _split(a):
    ah = a.astype(jnp.bfloat16)
    al = (a - ah.astype(jnp.float32)).astype(jnp.bfloat16)
    return ah, al


def _dot3(ah, al, bh, bl):
    d = lambda u, v: jax.lax.dot(u, v, preferred_element_type=jnp.float32)
    return d(ah, bh) + d(ah, bl) + d(al, bh)


def _dft_stats_kernel(x_ref, cmh_h_ref, cmh_l_ref, smh_h_ref, smh_l_ref,
                      cm_h_ref, cm_l_ref, sm_h_ref, sm_l_ref, out_ref):
    cmh = (cmh_h_ref[...], cmh_l_ref[...])
    smh = (smh_h_ref[...], smh_l_ref[...])
    cm = (cm_h_ref[...], cm_l_ref[...])
    sm = (sm_h_ref[...], sm_l_ref[...])
    alt = jnp.where(
        jax.lax.broadcasted_iota(jnp.int32, (_IMG, _IMG), 1) % 2 == 0,
        1.0, -1.0)
    lane = jax.lax.broadcasted_iota(jnp.int32, (3, 16), 1)
    rowi = jax.lax.broadcasted_iota(jnp.int32, (3, 16), 0)
    lane2 = jax.lax.broadcasted_iota(jnp.int32, (_IMG, 6), 1)

    gs, s1s, s2s, dcs = [], [], [], []
    for j in range(6):
        x = x_ref[j // 3, j % 3]
        xs = _split(x)
        p = _dot3(*xs, *cmh)
        q = _dot3(*xs, *smh)
        ps = _split(p)
        qs = _split(q)
        fre = _dot3(*cm, *ps) - _dot3(*sm, *qs)
        fim = -(_dot3(*cm, *qs) + _dot3(*sm, *ps))
        mag = jnp.sqrt(fre * fre + fim * fim)
        ang = jnp.arctan2(fim, fre)
        gs.append(jnp.sum(x * alt, axis=1, keepdims=True))
        s1s.append(jnp.sum(mag))
        s2s.append(jnp.sum(ang))
        dcs.append(jnp.sum(x))

    gcat = jnp.concatenate(gs, axis=1)
    gsp = _split(gcat)
    f128re = _dot3(*cm, *gsp)
    f128im = -_dot3(*sm, *gsp)
    mag128 = jnp.sqrt(f128re * f128re + f128im * f128im)
    ang128 = jnp.arctan2(f128im, f128re)
    for i in range(2):
        rows = []
        for jj in range(3):
            j = i * 3 + jj
            sel = (lane2 == j).astype(jnp.float32)
            s1 = s1s[j] + jnp.sum(mag128 * sel)
            s2 = s2s[j] + jnp.sum(ang128 * sel)
            rows.append(jnp.where(
                lane == 0, s1,
                jnp.where(lane == 1, s2, jnp.where(lane == 2, dcs[j], 0.0))))
        out_ref[i] = jnp.where(rowi == 0, rows[0],
                               jnp.where(rowi == 1, rows[1], rows[2]))


def _sigmoid_v(v):
    return 1.0 / (1.0 + jnp.exp(-v))


def _band_mask_v(featv, w1, b1, w2, b2):
    c1 = _sigmoid_v(featv * w1 + b1)
    c2 = _sigmoid_v(featv * w2 + b2)
    ind = jnp.where(c2 > c1, 1.0, 0.0)
    return jnp.where(c1 * ind < 1.0 / _IMG, 1.0, 0.0)


def _epilogue_sc_kernel(stats_hbm, wconst_hbm, out_hbm,
                        stats_v, wconst_v, row_v, row2_v,
                        sem_in, sem_r1, sem_r2):
    info = plsc.get_sparse_core_info()
    nc = info.num_cores
    wid = lax.axis_index("s") * nc + lax.axis_index("c")

    cp1 = pltpu.async_copy(stats_hbm, stats_v, sem_in)
    cp2 = pltpu.async_copy(wconst_hbm, wconst_v, sem_in)
    cp1.wait()
    cp2.wait()
    _HD = 256

    inv = jnp.float32(1.0 / (_IMG * _IMG))
    zeros16 = jnp.zeros((16,), jnp.int32)

    dnums = lax.GatherDimensionNumbers(
        offset_dims=(), collapsed_slice_dims=(0,), start_index_map=(0,))

    def splat(v, i):
        return lax.gather(v, (zeros16 + i)[:, None], dnums, (1,),
                          mode=lax.GatherScatterMode.PROMISE_IN_BOUNDS)

    def pooled(bc):
        srow = stats_v[pl.ds(bc * 16, 16)]
        feat1 = splat(srow, 0) * (1.0 / _NBINS)
        feat2 = splat(srow, 1) * (1.0 / _NBINS)
        dcv = splat(srow, 2)
        masks = []
        for m, featv in ((0, feat1), (1, feat2)):
            mrow = lambda a, k: wconst_v[pl.ds(((m * 2 + a) * 4 + k) * 16, 16)]
            mx = _band_mask_v(featv, mrow(0, 0), mrow(0, 1),
                              mrow(0, 2), mrow(0, 3))
            my = _band_mask_v(featv, mrow(1, 0), mrow(1, 1),
                              mrow(1, 2), mrow(1, 3))
            masks.append(jnp.minimum(mx + my, 1.0))
        mask1, mask2 = masks
        amp = jnp.abs(dcv) * inv
        negf = jnp.where(dcv < 0.0, 1.0, 0.0)
        cos_d = 1.0 - 2.0 * negf * mask2
        cos_c = 1.0 - 2.0 * negf * (1.0 - mask2)
        return amp * mask1 * cos_d, amp * (1.0 - mask1) * cos_c

    def emit_row(r, buf, sem):
        b = r // _NP
        n = r % _NP
        for c in range(3):
            pdrow, pcrow = pooled(b * 3 + c)
            pd = splat(pdrow, n)
            pc = splat(pcrow, n)
            for k in range(_FC // 16):
                ws = wconst_v[pl.ds(_HD + c * _FC + k * 16, 16)]
                wg = wconst_v[pl.ds(_HD + (4 + c) * _FC + k * 16, 16)]
                if c == 0:
                    bs = wconst_v[pl.ds(_HD + 3 * _FC + k * 16, 16)]
                    bg = wconst_v[pl.ds(_HD + 7 * _FC + k * 16, 16)]
                    buf[pl.ds(k * 16, 16)] = bs + pc * ws
                    buf[pl.ds(_FC + k * 16, 16)] = bg + pd * wg
                else:
                    buf[pl.ds(k * 16, 16)] = (
                        buf[pl.ds(k * 16, 16)] + pc * ws)
                    buf[pl.ds(_FC + k * 16, 16)] = (
                        buf[pl.ds(_FC + k * 16, 16)] + pd * wg)
        return pltpu.async_copy(buf, out_hbm.at[r], sem)

    d1 = emit_row(wid, row_v, sem_r1)

    @pl.when(wid < 8)
    def _():
        emit_row(wid + 32, row2_v, sem_r2).wait()

    d1.wait()


def kernel(x, W1, B1, W2, B2, Wsem, bsem, Wgen, bgen):
    B, C, W, H = x.shape

    idx = jnp.arange(_IMG, dtype=jnp.int32)
    m = (idx[:, None] * idx[None, :]) % _IMG
    theta = (2.0 * np.pi / _IMG) * m.astype(jnp.float32)
    cm = jnp.cos(theta)
    sm = jnp.sin(theta)
    cm_h, cm_l = _split(cm)
    sm_h, sm_l = _split(sm)

    full = lambda: pl.BlockSpec((_IMG, _IMG), lambda i: (0, 0))
    half = lambda: pl.BlockSpec((_IMG, 128), lambda i: (0, 0))
    stats = pl.pallas_call(
        _dft_stats_kernel,
        grid=(B // 2,),
        in_specs=[
            pl.BlockSpec((2, C, _IMG, _IMG), lambda i: (i, 0, 0, 0)),
            half(), half(), half(), half(),
            full(), full(), full(), full(),
        ],
        out_specs=pl.BlockSpec((2, 3, 16), lambda i: (i, 0, 0)),
        out_shape=jax.ShapeDtypeStruct((B, 3, 16), jnp.float32),
    )(x, cm_h[:, :128], cm_l[:, :128], sm_h[:, :128], sm_l[:, :128],
      cm_h, cm_l, sm_h, sm_l)
    stats_s = stats.reshape(-1)

    rows = []
    for Wm, Bm in ((W1, B1), (W2, B2)):
        for a in range(2):
            for arr in (Wm[a, 1], Bm[a, 1], Wm[a, 2], Bm[a, 2]):
                rows.append(jnp.pad(arr, (0, 16 - _NP)))
    wconst = jnp.concatenate(
        [jnp.stack(rows).reshape(-1), Wsem.reshape(-1), bsem,
         Wgen.reshape(-1), bgen])

    mesh = plsc.VectorSubcoreMesh(core_axis_name="c", subcore_axis_name="s")
    epilogue = pl.kernel(
        _epilogue_sc_kernel,
        mesh=mesh,
        out_type=jax.ShapeDtypeStruct((B * _NP, 2 * _FC), jnp.float32),
        scratch_types=[
            pltpu.VMEM((12 * 16,), jnp.float32),
            pltpu.VMEM((2304,), jnp.float32),
            pltpu.VMEM((2 * _FC,), jnp.float32),
            pltpu.VMEM((2 * _FC,), jnp.float32),
            pltpu.SemaphoreType.DMA,
            pltpu.SemaphoreType.DMA,
            pltpu.SemaphoreType.DMA,
        ],
    )
    return epilogue(stats_s, wconst)


# speedup vs baseline: 1.5202x; 1.0411x over previous
import jax
import jax.numpy as jnp
import ml_dtypes
import numpy as np
from jax import lax
from jax.experimental import pallas as pl
from jax.experimental.pallas import tpu as pltpu
from jax.experimental.pallas import tpu_sc as plsc

_NP = 10
_IMG = 256
_HALF = _IMG // 2 + 1
_NBINS = _IMG * _HALF
_FC = 256


def _split(a):
    ah = a.astype(jnp.bfloat16)
    al = (a - ah.astype(jnp.float32)).astype(jnp.bfloat16)
    return ah, al


def _dot3(ah, al, bh, bl):
    d = lambda u, v: jax.lax.dot(u, v, preferred_element_type=jnp.float32)
    return d(ah, bh) + d(ah, bl) + d(al, bh)


def _dft_stats_kernel(x_ref, cmh_h_ref, cmh_l_ref, smh_h_ref, smh_l_ref,
                      cm_h_ref, cm_l_ref, sm_h_ref, sm_l_ref, out_ref):
    cmh = (cmh_h_ref[...], cmh_l_ref[...])
    smh = (smh_h_ref[...], smh_l_ref[...])
    cm = (cm_h_ref[...], cm_l_ref[...])
    sm = (sm_h_ref[...], sm_l_ref[...])
    alt = jnp.where(
        jax.lax.broadcasted_iota(jnp.int32, (_IMG, _IMG), 1) % 2 == 0,
        1.0, -1.0)
    lane = jax.lax.broadcasted_iota(jnp.int32, (3, 16), 1)
    rowi = jax.lax.broadcasted_iota(jnp.int32, (3, 16), 0)
    lane2 = jax.lax.broadcasted_iota(jnp.int32, (_IMG, 6), 1)

    gs, s1s, s2s, dcs = [], [], [], []
    for j in range(6):
        x = x_ref[j // 3, j % 3]
        xs = _split(x)
        p = _dot3(*xs, *cmh)
        q = _dot3(*xs, *smh)
        ps = _split(p)
        qs = _split(q)
        fre = _dot3(*cm, *ps) - _dot3(*sm, *qs)
        fim = -(_dot3(*cm, *qs) + _dot3(*sm, *ps))
        mag = jnp.sqrt(fre * fre + fim * fim)
        ang = jnp.arctan2(fim, fre)
        gs.append(jnp.sum(x * alt, axis=1, keepdims=True))
        s1s.append(jnp.sum(mag))
        s2s.append(jnp.sum(ang))
        dcs.append(jnp.sum(x))

    gcat = jnp.concatenate(gs, axis=1)
    gsp = _split(gcat)
    f128re = _dot3(*cm, *gsp)
    f128im = -_dot3(*sm, *gsp)
    mag128 = jnp.sqrt(f128re * f128re + f128im * f128im)
    ang128 = jnp.arctan2(f128im, f128re)
    for i in range(2):
        rows = []
        for jj in range(3):
            j = i * 3 + jj
            sel = (lane2 == j).astype(jnp.float32)
            s1 = s1s[j] + jnp.sum(mag128 * sel)
            s2 = s2s[j] + jnp.sum(ang128 * sel)
            rows.append(jnp.where(
                lane == 0, s1,
                jnp.where(lane == 1, s2, jnp.where(lane == 2, dcs[j], 0.0))))
        out_ref[i] = jnp.where(rowi == 0, rows[0],
                               jnp.where(rowi == 1, rows[1], rows[2]))


def _sigmoid_v(v):
    return 1.0 / (1.0 + jnp.exp(-v))


def _band_mask_v(featv, w1, b1, w2, b2):
    c1 = _sigmoid_v(featv * w1 + b1)
    c2 = _sigmoid_v(featv * w2 + b2)
    ind = jnp.where(c2 > c1, 1.0, 0.0)
    return jnp.where(c1 * ind < 1.0 / _IMG, 1.0, 0.0)


def _epilogue_sc_kernel(stats_hbm, wconst_hbm, out_hbm,
                        stats_v, wconst_v, row_v, row2_v,
                        sem_in, sem_r1, sem_r2):
    info = plsc.get_sparse_core_info()
    nc = info.num_cores
    wid = lax.axis_index("s") * nc + lax.axis_index("c")

    cp1 = pltpu.async_copy(stats_hbm, stats_v, sem_in)
    cp2 = pltpu.async_copy(wconst_hbm, wconst_v, sem_in)
    cp1.wait()
    cp2.wait()
    _HD = 256

    inv = jnp.float32(1.0 / (_IMG * _IMG))
    zeros16 = jnp.zeros((16,), jnp.int32)

    dnums = lax.GatherDimensionNumbers(
        offset_dims=(), collapsed_slice_dims=(0,), start_index_map=(0,))

    def splat(v, i):
        return lax.gather(v, (zeros16 + i)[:, None], dnums, (1,),
                          mode=lax.GatherScatterMode.PROMISE_IN_BOUNDS)

    def pooled(bc):
        srow = stats_v[pl.ds(bc * 16, 16)]
        feat1 = splat(srow, 0) * (1.0 / _NBINS)
        feat2 = splat(srow, 1) * (1.0 / _NBINS)
        dcv = splat(srow, 2)
        masks = []
        for m, featv in ((0, feat1), (1, feat2)):
            mrow = lambda a, k: wconst_v[pl.ds(((m * 2 + a) * 4 + k) * 16, 16)]
            mx = _band_mask_v(featv, mrow(0, 0), mrow(0, 1),
                              mrow(0, 2), mrow(0, 3))
            my = _band_mask_v(featv, mrow(1, 0), mrow(1, 1),
                              mrow(1, 2), mrow(1, 3))
            masks.append(jnp.minimum(mx + my, 1.0))
        mask1, mask2 = masks
        amp = jnp.abs(dcv) * inv
        negf = jnp.where(dcv < 0.0, 1.0, 0.0)
        cos_d = 1.0 - 2.0 * negf * mask2
        cos_c = 1.0 - 2.0 * negf * (1.0 - mask2)
        return amp * mask1 * cos_d, amp * (1.0 - mask1) * cos_c

    def emit_row(r, buf, sem):
        b = r // _NP
        n = r % _NP
        for c in range(3):
            pdrow, pcrow = pooled(b * 3 + c)
            pd = splat(pdrow, n)
            pc = splat(pcrow, n)
            for k in range(_FC // 16):
                ws = wconst_v[pl.ds(_HD + c * _FC + k * 16, 16)]
                wg = wconst_v[pl.ds(_HD + (4 + c) * _FC + k * 16, 16)]
                if c == 0:
                    bs = wconst_v[pl.ds(_HD + 3 * _FC + k * 16, 16)]
                    bg = wconst_v[pl.ds(_HD + 7 * _FC + k * 16, 16)]
                    buf[pl.ds(k * 16, 16)] = bs + pc * ws
                    buf[pl.ds(_FC + k * 16, 16)] = bg + pd * wg
                else:
                    buf[pl.ds(k * 16, 16)] = (
                        buf[pl.ds(k * 16, 16)] + pc * ws)
                    buf[pl.ds(_FC + k * 16, 16)] = (
                        buf[pl.ds(_FC + k * 16, 16)] + pd * wg)
        return pltpu.async_copy(buf, out_hbm.at[r], sem)

    d1 = emit_row(wid, row_v, sem_r1)

    @pl.when(wid < 8)
    def _():
        emit_row(wid + 32, row2_v, sem_r2).wait()

    d1.wait()


def kernel(x, W1, B1, W2, B2, Wsem, bsem, Wgen, bgen):
    B, C, W, H = x.shape

    idx = np.arange(_IMG, dtype=np.int64)
    theta = (2.0 * np.pi / _IMG) * (np.outer(idx, idx) % _IMG)

    def _np_split(a):
        a = a.astype(np.float32)
        ah = a.astype(ml_dtypes.bfloat16)
        al = (a - ah.astype(np.float32)).astype(ml_dtypes.bfloat16)
        return ah, al

    cm_h, cm_l = _np_split(np.cos(theta))
    sm_h, sm_l = _np_split(np.sin(theta))

    full = lambda: pl.BlockSpec((_IMG, _IMG), lambda i: (0, 0))
    half = lambda: pl.BlockSpec((_IMG, 128), lambda i: (0, 0))
    stats = pl.pallas_call(
        _dft_stats_kernel,
        grid=(B // 2,),
        in_specs=[
            pl.BlockSpec((2, C, _IMG, _IMG), lambda i: (i, 0, 0, 0)),
            half(), half(), half(), half(),
            full(), full(), full(), full(),
        ],
        out_specs=pl.BlockSpec((2, 3, 16), lambda i: (i, 0, 0)),
        out_shape=jax.ShapeDtypeStruct((B, 3, 16), jnp.float32),
    )(x,
      jnp.asarray(cm_h[:, :128]), jnp.asarray(cm_l[:, :128]),
      jnp.asarray(sm_h[:, :128]), jnp.asarray(sm_l[:, :128]),
      jnp.asarray(cm_h), jnp.asarray(cm_l),
      jnp.asarray(sm_h), jnp.asarray(sm_l))
    stats_s = stats.reshape(-1)

    rows = []
    for Wm, Bm in ((W1, B1), (W2, B2)):
        for a in range(2):
            for arr in (Wm[a, 1], Bm[a, 1], Wm[a, 2], Bm[a, 2]):
                rows.append(jnp.pad(arr, (0, 16 - _NP)))
    wconst = jnp.concatenate(
        [jnp.stack(rows).reshape(-1), Wsem.reshape(-1), bsem,
         Wgen.reshape(-1), bgen])

    mesh = plsc.VectorSubcoreMesh(core_axis_name="c", subcore_axis_name="s")
    epilogue = pl.kernel(
        _epilogue_sc_kernel,
        mesh=mesh,
        out_type=jax.ShapeDtypeStruct((B * _NP, 2 * _FC), jnp.float32),
        scratch_types=[
            pltpu.VMEM((12 * 16,), jnp.float32),
            pltpu.VMEM((2304,), jnp.float32),
            pltpu.VMEM((2 * _FC,), jnp.float32),
            pltpu.VMEM((2 * _FC,), jnp.float32),
            pltpu.SemaphoreType.DMA,
            pltpu.SemaphoreType.DMA,
            pltpu.SemaphoreType.DMA,
        ],
    )
    return epilogue(stats_s, wconst)


# speedup vs baseline: 1.5206x; 1.0002x over previous
import jax
import jax.numpy as jnp
import ml_dtypes
import numpy as np
from jax import lax
from jax.experimental import pallas as pl
from jax.experimental.pallas import tpu as pltpu
from jax.experimental.pallas import tpu_sc as plsc

_NP = 10
_IMG = 256
_HALF = _IMG // 2 + 1
_NBINS = _IMG * _HALF
_FC = 256


def _split(a):
    ah = a.astype(jnp.bfloat16)
    al = (a - ah.astype(jnp.float32)).astype(jnp.bfloat16)
    return ah, al


def _dot3(ah, al, bh, bl):
    d = lambda u, v: jax.lax.dot(u, v, preferred_element_type=jnp.float32)
    return d(ah, bh) + d(ah, bl) + d(al, bh)


def _dft_stats_kernel(x_ref, cm_h_ref, cm_l_ref, sm_h_ref, sm_l_ref,
                      out_ref):
    cm = (cm_h_ref[...], cm_l_ref[...])
    sm = (sm_h_ref[...], sm_l_ref[...])
    cmh = (cm[0][:, :128], cm[1][:, :128])
    smh = (sm[0][:, :128], sm[1][:, :128])
    alt = jnp.where(
        jax.lax.broadcasted_iota(jnp.int32, (_IMG, _IMG), 1) % 2 == 0,
        1.0, -1.0)
    lane = jax.lax.broadcasted_iota(jnp.int32, (3, 16), 1)
    rowi = jax.lax.broadcasted_iota(jnp.int32, (3, 16), 0)
    lane2 = jax.lax.broadcasted_iota(jnp.int32, (_IMG, 6), 1)

    gs, s1s, s2s, dcs = [], [], [], []
    for j in range(6):
        x = x_ref[j // 3, j % 3]
        xs = _split(x)
        p = _dot3(*xs, *cmh)
        q = _dot3(*xs, *smh)
        ps = _split(p)
        qs = _split(q)
        fre = _dot3(*cm, *ps) - _dot3(*sm, *qs)
        fim = -(_dot3(*cm, *qs) + _dot3(*sm, *ps))
        mag = jnp.sqrt(fre * fre + fim * fim)
        ang = jnp.arctan2(fim, fre)
        gs.append(jnp.sum(x * alt, axis=1, keepdims=True))
        s1s.append(jnp.sum(mag))
        s2s.append(jnp.sum(ang))
        dcs.append(jnp.sum(x))

    gcat = jnp.concatenate(gs, axis=1)
    gsp = _split(gcat)
    f128re = _dot3(*cm, *gsp)
    f128im = -_dot3(*sm, *gsp)
    mag128 = jnp.sqrt(f128re * f128re + f128im * f128im)
    ang128 = jnp.arctan2(f128im, f128re)
    for i in range(2):
        rows = []
        for jj in range(3):
            j = i * 3 + jj
            sel = (lane2 == j).astype(jnp.float32)
            s1 = s1s[j] + jnp.sum(mag128 * sel)
            s2 = s2s[j] + jnp.sum(ang128 * sel)
            rows.append(jnp.where(
                lane == 0, s1,
                jnp.where(lane == 1, s2, jnp.where(lane == 2, dcs[j], 0.0))))
        out_ref[i] = jnp.where(rowi == 0, rows[0],
                               jnp.where(rowi == 1, rows[1], rows[2]))


def _sigmoid_v(v):
    return 1.0 / (1.0 + jnp.exp(-v))


def _band_mask_v(featv, w1, b1, w2, b2):
    c1 = _sigmoid_v(featv * w1 + b1)
    c2 = _sigmoid_v(featv * w2 + b2)
    ind = jnp.where(c2 > c1, 1.0, 0.0)
    return jnp.where(c1 * ind < 1.0 / _IMG, 1.0, 0.0)


def _epilogue_sc_kernel(stats_hbm, wconst_hbm, out_hbm,
                        stats_v, wconst_v, row_v, row2_v,
                        sem_in, sem_r1, sem_r2):
    info = plsc.get_sparse_core_info()
    nc = info.num_cores
    wid = lax.axis_index("s") * nc + lax.axis_index("c")

    cp1 = pltpu.async_copy(stats_hbm, stats_v, sem_in)
    cp2 = pltpu.async_copy(wconst_hbm, wconst_v, sem_in)
    cp1.wait()
    cp2.wait()
    _HD = 256

    inv = jnp.float32(1.0 / (_IMG * _IMG))
    zeros16 = jnp.zeros((16,), jnp.int32)

    dnums = lax.GatherDimensionNumbers(
        offset_dims=(), collapsed_slice_dims=(0,), start_index_map=(0,))

    def splat(v, i):
        return lax.gather(v, (zeros16 + i)[:, None], dnums, (1,),
                          mode=lax.GatherScatterMode.PROMISE_IN_BOUNDS)

    def pooled(bc):
        srow = stats_v[pl.ds(bc * 16, 16)]
        feat1 = splat(srow, 0) * (1.0 / _NBINS)
        feat2 = splat(srow, 1) * (1.0 / _NBINS)
        dcv = splat(srow, 2)
        masks = []
        for m, featv in ((0, feat1), (1, feat2)):
            mrow = lambda a, k: wconst_v[pl.ds(((m * 2 + a) * 4 + k) * 16, 16)]
            mx = _band_mask_v(featv, mrow(0, 0), mrow(0, 1),
                              mrow(0, 2), mrow(0, 3))
            my = _band_mask_v(featv, mrow(1, 0), mrow(1, 1),
                              mrow(1, 2), mrow(1, 3))
            masks.append(jnp.minimum(mx + my, 1.0))
        mask1, mask2 = masks
        amp = jnp.abs(dcv) * inv
        negf = jnp.where(dcv < 0.0, 1.0, 0.0)
        cos_d = 1.0 - 2.0 * negf * mask2
        cos_c = 1.0 - 2.0 * negf * (1.0 - mask2)
        return amp * mask1 * cos_d, amp * (1.0 - mask1) * cos_c

    def emit_row(r, buf, sem):
        b = r // _NP
        n = r % _NP
        for c in range(3):
            pdrow, pcrow = pooled(b * 3 + c)
            pd = splat(pdrow, n)
            pc = splat(pcrow, n)
            for k in range(_FC // 16):
                ws = wconst_v[pl.ds(_HD + c * _FC + k * 16, 16)]
                wg = wconst_v[pl.ds(_HD + (4 + c) * _FC + k * 16, 16)]
                if c == 0:
                    bs = wconst_v[pl.ds(_HD + 3 * _FC + k * 16, 16)]
                    bg = wconst_v[pl.ds(_HD + 7 * _FC + k * 16, 16)]
                    buf[pl.ds(k * 16, 16)] = bs + pc * ws
                    buf[pl.ds(_FC + k * 16, 16)] = bg + pd * wg
                else:
                    buf[pl.ds(k * 16, 16)] = (
                        buf[pl.ds(k * 16, 16)] + pc * ws)
                    buf[pl.ds(_FC + k * 16, 16)] = (
                        buf[pl.ds(_FC + k * 16, 16)] + pd * wg)
        return pltpu.async_copy(buf, out_hbm.at[r], sem)

    d1 = emit_row(wid, row_v, sem_r1)

    @pl.when(wid < 8)
    def _():
        emit_row(wid + 32, row2_v, sem_r2).wait()

    d1.wait()


def kernel(x, W1, B1, W2, B2, Wsem, bsem, Wgen, bgen):
    B, C, W, H = x.shape

    idx = np.arange(_IMG, dtype=np.int64)
    theta = (2.0 * np.pi / _IMG) * (np.outer(idx, idx) % _IMG)

    def _np_split(a):
        a = a.astype(np.float32)
        ah = a.astype(ml_dtypes.bfloat16)
        al = (a - ah.astype(np.float32)).astype(ml_dtypes.bfloat16)
        return ah, al

    cm_h, cm_l = _np_split(np.cos(theta))
    sm_h, sm_l = _np_split(np.sin(theta))

    full = lambda: pl.BlockSpec((_IMG, _IMG), lambda i: (0, 0))
    stats = pl.pallas_call(
        _dft_stats_kernel,
        grid=(B // 2,),
        in_specs=[
            pl.BlockSpec((2, C, _IMG, _IMG), lambda i: (i, 0, 0, 0)),
            full(), full(), full(), full(),
        ],
        out_specs=pl.BlockSpec((2, 3, 16), lambda i: (i, 0, 0)),
        out_shape=jax.ShapeDtypeStruct((B, 3, 16), jnp.float32),
    )(x, jnp.asarray(cm_h), jnp.asarray(cm_l),
      jnp.asarray(sm_h), jnp.asarray(sm_l))
    stats_s = stats.reshape(-1)

    rows = []
    for Wm, Bm in ((W1, B1), (W2, B2)):
        for a in range(2):
            for arr in (Wm[a, 1], Bm[a, 1], Wm[a, 2], Bm[a, 2]):
                rows.append(jnp.pad(arr, (0, 16 - _NP)))
    wconst = jnp.concatenate(
        [jnp.stack(rows).reshape(-1), Wsem.reshape(-1), bsem,
         Wgen.reshape(-1), bgen])

    mesh = plsc.VectorSubcoreMesh(core_axis_name="c", subcore_axis_name="s")
    epilogue = pl.kernel(
        _epilogue_sc_kernel,
        mesh=mesh,
        out_type=jax.ShapeDtypeStruct((B * _NP, 2 * _FC), jnp.float32),
        scratch_types=[
            pltpu.VMEM((12 * 16,), jnp.float32),
            pltpu.VMEM((2304,), jnp.float32),
            pltpu.VMEM((2 * _FC,), jnp.float32),
            pltpu.VMEM((2 * _FC,), jnp.float32),
            pltpu.SemaphoreType.DMA,
            pltpu.SemaphoreType.DMA,
            pltpu.SemaphoreType.DMA,
        ],
    )
    return epilogue(stats_s, wconst)


# speedup vs baseline: 1.5208x; 1.0002x over previous
import jax
import jax.numpy as jnp
import ml_dtypes
import numpy as np
from jax import lax
from jax.experimental import pallas as pl
from jax.experimental.pallas import tpu as pltpu
from jax.experimental.pallas import tpu_sc as plsc

_NP = 10
_IMG = 256
_HALF = _IMG // 2 + 1
_NBINS = _IMG * _HALF
_FC = 256


def _split(a):
    ah = a.astype(jnp.bfloat16)
    al = (a - ah.astype(jnp.float32)).astype(jnp.bfloat16)
    return ah, al


def _dot3(ah, al, bh, bl):
    d = lambda u, v: jax.lax.dot(u, v, preferred_element_type=jnp.float32)
    return d(ah, bh) + d(ah, bl) + d(al, bh)


def _dft_stats_kernel(x_ref, tab_ref, out_ref):
    cm = (tab_ref[0], tab_ref[1])
    sm = (tab_ref[2], tab_ref[3])
    cmh = (cm[0][:, :128], cm[1][:, :128])
    smh = (sm[0][:, :128], sm[1][:, :128])
    alt = jnp.where(
        jax.lax.broadcasted_iota(jnp.int32, (_IMG, _IMG), 1) % 2 == 0,
        1.0, -1.0)
    lane = jax.lax.broadcasted_iota(jnp.int32, (3, 16), 1)
    rowi = jax.lax.broadcasted_iota(jnp.int32, (3, 16), 0)
    lane2 = jax.lax.broadcasted_iota(jnp.int32, (_IMG, 6), 1)

    gs, s1s, s2s, dcs = [], [], [], []
    for j in range(6):
        x = x_ref[j // 3, j % 3]
        xs = _split(x)
        p = _dot3(*xs, *cmh)
        q = _dot3(*xs, *smh)
        ps = _split(p)
        qs = _split(q)
        fre = _dot3(*cm, *ps) - _dot3(*sm, *qs)
        fim = -(_dot3(*cm, *qs) + _dot3(*sm, *ps))
        mag = jnp.sqrt(fre * fre + fim * fim)
        ang = jnp.arctan2(fim, fre)
        gs.append(jnp.sum(x * alt, axis=1, keepdims=True))
        s1s.append(jnp.sum(mag))
        s2s.append(jnp.sum(ang))
        dcs.append(jnp.sum(x))

    gcat = jnp.concatenate(gs, axis=1)
    gsp = _split(gcat)
    f128re = _dot3(*cm, *gsp)
    f128im = -_dot3(*sm, *gsp)
    mag128 = jnp.sqrt(f128re * f128re + f128im * f128im)
    ang128 = jnp.arctan2(f128im, f128re)
    for i in range(2):
        rows = []
        for jj in range(3):
            j = i * 3 + jj
            sel = (lane2 == j).astype(jnp.float32)
            s1 = s1s[j] + jnp.sum(mag128 * sel)
            s2 = s2s[j] + jnp.sum(ang128 * sel)
            rows.append(jnp.where(
                lane == 0, s1,
                jnp.where(lane == 1, s2, jnp.where(lane == 2, dcs[j], 0.0))))
        out_ref[i] = jnp.where(rowi == 0, rows[0],
                               jnp.where(rowi == 1, rows[1], rows[2]))


def _sigmoid_v(v):
    return 1.0 / (1.0 + jnp.exp(-v))


def _band_mask_v(featv, w1, b1, w2, b2):
    c1 = _sigmoid_v(featv * w1 + b1)
    c2 = _sigmoid_v(featv * w2 + b2)
    ind = jnp.where(c2 > c1, 1.0, 0.0)
    return jnp.where(c1 * ind < 1.0 / _IMG, 1.0, 0.0)


def _epilogue_sc_kernel(stats_hbm, wconst_hbm, out_hbm,
                        stats_v, wconst_v, row_v, row2_v,
                        sem_in, sem_r1, sem_r2):
    info = plsc.get_sparse_core_info()
    nc = info.num_cores
    wid = lax.axis_index("s") * nc + lax.axis_index("c")

    cp1 = pltpu.async_copy(stats_hbm, stats_v, sem_in)
    cp2 = pltpu.async_copy(wconst_hbm, wconst_v, sem_in)
    cp1.wait()
    cp2.wait()
    _HD = 256

    inv = jnp.float32(1.0 / (_IMG * _IMG))
    zeros16 = jnp.zeros((16,), jnp.int32)

    dnums = lax.GatherDimensionNumbers(
        offset_dims=(), collapsed_slice_dims=(0,), start_index_map=(0,))

    def splat(v, i):
        return lax.gather(v, (zeros16 + i)[:, None], dnums, (1,),
                          mode=lax.GatherScatterMode.PROMISE_IN_BOUNDS)

    def pooled(bc):
        srow = stats_v[pl.ds(bc * 16, 16)]
        feat1 = splat(srow, 0) * (1.0 / _NBINS)
        feat2 = splat(srow, 1) * (1.0 / _NBINS)
        dcv = splat(srow, 2)
        masks = []
        for m, featv in ((0, feat1), (1, feat2)):
            mrow = lambda a, k: wconst_v[pl.ds(((m * 2 + a) * 4 + k) * 16, 16)]
            mx = _band_mask_v(featv, mrow(0, 0), mrow(0, 1),
                              mrow(0, 2), mrow(0, 3))
            my = _band_mask_v(featv, mrow(1, 0), mrow(1, 1),
                              mrow(1, 2), mrow(1, 3))
            masks.append(jnp.minimum(mx + my, 1.0))
        mask1, mask2 = masks
        amp = jnp.abs(dcv) * inv
        negf = jnp.where(dcv < 0.0, 1.0, 0.0)
        cos_d = 1.0 - 2.0 * negf * mask2
        cos_c = 1.0 - 2.0 * negf * (1.0 - mask2)
        return amp * mask1 * cos_d, amp * (1.0 - mask1) * cos_c

    def emit_row(r, buf, sem):
        b = r // _NP
        n = r % _NP
        for c in range(3):
            pdrow, pcrow = pooled(b * 3 + c)
            pd = splat(pdrow, n)
            pc = splat(pcrow, n)
            for k in range(_FC // 16):
                ws = wconst_v[pl.ds(_HD + c * _FC + k * 16, 16)]
                wg = wconst_v[pl.ds(_HD + (4 + c) * _FC + k * 16, 16)]
                if c == 0:
                    bs = wconst_v[pl.ds(_HD + 3 * _FC + k * 16, 16)]
                    bg = wconst_v[pl.ds(_HD + 7 * _FC + k * 16, 16)]
                    buf[pl.ds(k * 16, 16)] = bs + pc * ws
                    buf[pl.ds(_FC + k * 16, 16)] = bg + pd * wg
                else:
                    buf[pl.ds(k * 16, 16)] = (
                        buf[pl.ds(k * 16, 16)] + pc * ws)
                    buf[pl.ds(_FC + k * 16, 16)] = (
                        buf[pl.ds(_FC + k * 16, 16)] + pd * wg)
        return pltpu.async_copy(buf, out_hbm.at[r], sem)

    d1 = emit_row(wid, row_v, sem_r1)

    @pl.when(wid < 8)
    def _():
        emit_row(wid + 32, row2_v, sem_r2).wait()

    d1.wait()


def kernel(x, W1, B1, W2, B2, Wsem, bsem, Wgen, bgen):
    B, C, W, H = x.shape

    idx = np.arange(_IMG, dtype=np.int64)
    theta = (2.0 * np.pi / _IMG) * (np.outer(idx, idx) % _IMG)

    def _np_split(a):
        a = a.astype(np.float32)
        ah = a.astype(ml_dtypes.bfloat16)
        al = (a - ah.astype(np.float32)).astype(ml_dtypes.bfloat16)
        return ah, al

    tab = np.stack([*_np_split(np.cos(theta)), *_np_split(np.sin(theta))])

    stats = pl.pallas_call(
        _dft_stats_kernel,
        grid=(B // 2,),
        in_specs=[
            pl.BlockSpec((2, C, _IMG, _IMG), lambda i: (i, 0, 0, 0)),
            pl.BlockSpec((4, _IMG, _IMG), lambda i: (0, 0, 0)),
        ],
        out_specs=pl.BlockSpec((2, 3, 16), lambda i: (i, 0, 0)),
        out_shape=jax.ShapeDtypeStruct((B, 3, 16), jnp.float32),
    )(x, jnp.asarray(tab))
    stats_s = stats.reshape(-1)

    rows = []
    for Wm, Bm in ((W1, B1), (W2, B2)):
        for a in range(2):
            for arr in (Wm[a, 1], Bm[a, 1], Wm[a, 2], Bm[a, 2]):
                rows.append(jnp.pad(arr, (0, 16 - _NP)))
    wconst = jnp.concatenate(
        [jnp.stack(rows).reshape(-1), Wsem.reshape(-1), bsem,
         Wgen.reshape(-1), bgen])

    mesh = plsc.VectorSubcoreMesh(core_axis_name="c", subcore_axis_name="s")
    epilogue = pl.kernel(
        _epilogue_sc_kernel,
        mesh=mesh,
        out_type=jax.ShapeDtypeStruct((B * _NP, 2 * _FC), jnp.float32),
        scratch_types=[
            pltpu.VMEM((12 * 16,), jnp.float32),
            pltpu.VMEM((2304,), jnp.float32),
            pltpu.VMEM((2 * _FC,), jnp.float32),
            pltpu.VMEM((2 * _FC,), jnp.float32),
            pltpu.SemaphoreType.DMA,
            pltpu.SemaphoreType.DMA,
            pltpu.SemaphoreType.DMA,
        ],
    )
    return epilogue(stats_s, wconst)


# speedup vs baseline: 1.5294x; 1.0056x over previous
import jax
import jax.numpy as jnp
import ml_dtypes
import numpy as np
from jax import lax
from jax.experimental import pallas as pl
from jax.experimental.pallas import tpu as pltpu
from jax.experimental.pallas import tpu_sc as plsc

_NP = 10
_IMG = 256
_HALF = _IMG // 2 + 1
_NBINS = _IMG * _HALF
_FC = 256


def _split(a):
    ah = a.astype(jnp.bfloat16)
    al = (a - ah.astype(jnp.float32)).astype(jnp.bfloat16)
    return ah, al


def _dot3(ah, al, bh, bl):
    d = lambda u, v: jax.lax.dot(u, v, preferred_element_type=jnp.float32)
    return d(ah, bh) + d(ah, bl) + d(al, bh)


def _dft_stats_kernel(x_ref, tab_ref, out_ref):
    cm = (tab_ref[0], tab_ref[1])
    sm = (tab_ref[2], tab_ref[3])
    cmh = (cm[0][:, :128], cm[1][:, :128])
    smh = (sm[0][:, :128], sm[1][:, :128])
    alt = jnp.where(
        jax.lax.broadcasted_iota(jnp.int32, (_IMG, _IMG), 1) % 2 == 0,
        1.0, -1.0)
    lane = jax.lax.broadcasted_iota(jnp.int32, (3, 16), 1)
    rowi = jax.lax.broadcasted_iota(jnp.int32, (3, 16), 0)
    lane2 = jax.lax.broadcasted_iota(jnp.int32, (_IMG, 12), 1)

    gs, s1s, s2s, dcs = [], [], [], []
    for j in range(12):
        x = x_ref[j // 3, j % 3]
        xs = _split(x)
        p = _dot3(*xs, *cmh)
        q = _dot3(*xs, *smh)
        ps = _split(p)
        qs = _split(q)
        fre = _dot3(*cm, *ps) - _dot3(*sm, *qs)
        fim = -(_dot3(*cm, *qs) + _dot3(*sm, *ps))
        mag = jnp.sqrt(fre * fre + fim * fim)
        ang = jnp.arctan2(fim, fre)
        gs.append(jnp.sum(x * alt, axis=1, keepdims=True))
        s1s.append(jnp.sum(mag))
        s2s.append(jnp.sum(ang))
        dcs.append(jnp.sum(x))

    gcat = jnp.concatenate(gs, axis=1)
    gsp = _split(gcat)
    f128re = _dot3(*cm, *gsp)
    f128im = -_dot3(*sm, *gsp)
    mag128 = jnp.sqrt(f128re * f128re + f128im * f128im)
    ang128 = jnp.arctan2(f128im, f128re)
    for i in range(4):
        rows = []
        for jj in range(3):
            j = i * 3 + jj
            sel = (lane2 == j).astype(jnp.float32)
            s1 = s1s[j] + jnp.sum(mag128 * sel)
            s2 = s2s[j] + jnp.sum(ang128 * sel)
            rows.append(jnp.where(
                lane == 0, s1,
                jnp.where(lane == 1, s2, jnp.where(lane == 2, dcs[j], 0.0))))
        out_ref[i] = jnp.where(rowi == 0, rows[0],
                               jnp.where(rowi == 1, rows[1], rows[2]))


def _sigmoid_v(v):
    return 1.0 / (1.0 + jnp.exp(-v))


def _band_mask_v(featv, w1, b1, w2, b2):
    c1 = _sigmoid_v(featv * w1 + b1)
    c2 = _sigmoid_v(featv * w2 + b2)
    ind = jnp.where(c2 > c1, 1.0, 0.0)
    return jnp.where(c1 * ind < 1.0 / _IMG, 1.0, 0.0)


def _epilogue_sc_kernel(stats_hbm, wconst_hbm, out_hbm,
                        stats_v, wconst_v, row_v, row2_v,
                        sem_in, sem_r1, sem_r2):
    info = plsc.get_sparse_core_info()
    nc = info.num_cores
    wid = lax.axis_index("s") * nc + lax.axis_index("c")

    cp1 = pltpu.async_copy(stats_hbm, stats_v, sem_in)
    cp2 = pltpu.async_copy(wconst_hbm, wconst_v, sem_in)
    cp1.wait()
    cp2.wait()
    _HD = 256

    inv = jnp.float32(1.0 / (_IMG * _IMG))
    zeros16 = jnp.zeros((16,), jnp.int32)

    dnums = lax.GatherDimensionNumbers(
        offset_dims=(), collapsed_slice_dims=(0,), start_index_map=(0,))

    def splat(v, i):
        return lax.gather(v, (zeros16 + i)[:, None], dnums, (1,),
                          mode=lax.GatherScatterMode.PROMISE_IN_BOUNDS)

    def pooled(bc):
        srow = stats_v[pl.ds(bc * 16, 16)]
        feat1 = splat(srow, 0) * (1.0 / _NBINS)
        feat2 = splat(srow, 1) * (1.0 / _NBINS)
        dcv = splat(srow, 2)
        masks = []
        for m, featv in ((0, feat1), (1, feat2)):
            mrow = lambda a, k: wconst_v[pl.ds(((m * 2 + a) * 4 + k) * 16, 16)]
            mx = _band_mask_v(featv, mrow(0, 0), mrow(0, 1),
                              mrow(0, 2), mrow(0, 3))
            my = _band_mask_v(featv, mrow(1, 0), mrow(1, 1),
                              mrow(1, 2), mrow(1, 3))
            masks.append(jnp.minimum(mx + my, 1.0))
        mask1, mask2 = masks
        amp = jnp.abs(dcv) * inv
        negf = jnp.where(dcv < 0.0, 1.0, 0.0)
        cos_d = 1.0 - 2.0 * negf * mask2
        cos_c = 1.0 - 2.0 * negf * (1.0 - mask2)
        return amp * mask1 * cos_d, amp * (1.0 - mask1) * cos_c

    def emit_row(r, buf, sem):
        b = r // _NP
        n = r % _NP
        for c in range(3):
            pdrow, pcrow = pooled(b * 3 + c)
            pd = splat(pdrow, n)
            pc = splat(pcrow, n)
            for k in range(_FC // 16):
                ws = wconst_v[pl.ds(_HD + c * _FC + k * 16, 16)]
                wg = wconst_v[pl.ds(_HD + (4 + c) * _FC + k * 16, 16)]
                if c == 0:
                    bs = wconst_v[pl.ds(_HD + 3 * _FC + k * 16, 16)]
                    bg = wconst_v[pl.ds(_HD + 7 * _FC + k * 16, 16)]
                    buf[pl.ds(k * 16, 16)] = bs + pc * ws
                    buf[pl.ds(_FC + k * 16, 16)] = bg + pd * wg
                else:
                    buf[pl.ds(k * 16, 16)] = (
                        buf[pl.ds(k * 16, 16)] + pc * ws)
                    buf[pl.ds(_FC + k * 16, 16)] = (
                        buf[pl.ds(_FC + k * 16, 16)] + pd * wg)
        return pltpu.async_copy(buf, out_hbm.at[r], sem)

    d1 = emit_row(wid, row_v, sem_r1)

    @pl.when(wid < 8)
    def _():
        emit_row(wid + 32, row2_v, sem_r2).wait()

    d1.wait()


def kernel(x, W1, B1, W2, B2, Wsem, bsem, Wgen, bgen):
    B, C, W, H = x.shape

    idx = np.arange(_IMG, dtype=np.int64)
    theta = (2.0 * np.pi / _IMG) * (np.outer(idx, idx) % _IMG)

    def _np_split(a):
        a = a.astype(np.float32)
        ah = a.astype(ml_dtypes.bfloat16)
        al = (a - ah.astype(np.float32)).astype(ml_dtypes.bfloat16)
        return ah, al

    tab = np.stack([*_np_split(np.cos(theta)), *_np_split(np.sin(theta))])

    stats = pl.pallas_call(
        _dft_stats_kernel,
        grid=(1,),
        in_specs=[
            pl.BlockSpec((B, C, _IMG, _IMG), lambda i: (0, 0, 0, 0)),
            pl.BlockSpec((4, _IMG, _IMG), lambda i: (0, 0, 0)),
        ],
        out_specs=pl.BlockSpec((B, 3, 16), lambda i: (0, 0, 0)),
        out_shape=jax.ShapeDtypeStruct((B, 3, 16), jnp.float32),
    )(x, jnp.asarray(tab))
    stats_s = stats.reshape(-1)

    rows = []
    for Wm, Bm in ((W1, B1), (W2, B2)):
        for a in range(2):
            for arr in (Wm[a, 1], Bm[a, 1], Wm[a, 2], Bm[a, 2]):
                rows.append(jnp.pad(arr, (0, 16 - _NP)))
    wconst = jnp.concatenate(
        [jnp.stack(rows).reshape(-1), Wsem.reshape(-1), bsem,
         Wgen.reshape(-1), bgen])

    mesh = plsc.VectorSubcoreMesh(core_axis_name="c", subcore_axis_name="s")
    epilogue = pl.kernel(
        _epilogue_sc_kernel,
        mesh=mesh,
        out_type=jax.ShapeDtypeStruct((B * _NP, 2 * _FC), jnp.float32),
        scratch_types=[
            pltpu.VMEM((12 * 16,), jnp.float32),
            pltpu.VMEM((2304,), jnp.float32),
            pltpu.VMEM((2 * _FC,), jnp.float32),
            pltpu.VMEM((2 * _FC,), jnp.float32),
            pltpu.SemaphoreType.DMA,
            pltpu.SemaphoreType.DMA,
            pltpu.SemaphoreType.DMA,
        ],
    )
    return epilogue(stats_s, wconst)
